# Initial kernel scaffold; baseline (speedup 1.0000x reference)
import jax
import jax.numpy as jnp
from jax.experimental import pallas as pl


def kernel(x, W10, b10, W11, b11, Wr0, br0, Wr1, br1, A1, c1, A2, c2, A3, c3, A4, c4, edge2, edge2_r, ei, pred_links, num_node, pos_mask):
    raise NotImplementedError("write your pallas kernel here")


# speedup vs baseline: 2.0629x; 2.0629x over previous
import functools

import jax
import jax.numpy as jnp
from jax.experimental import pallas as pl
from jax.experimental.pallas import tpu as pltpu

N_NODE = 10000


def _ln(x, eps=1e-5):
    mu = jnp.mean(x, axis=-1, keepdims=True)
    var = jnp.mean((x - mu) ** 2, axis=-1, keepdims=True)
    return (x - mu) / jnp.sqrt(var + eps)


def _gcn_structure(edge, N):
    row = edge[0]
    col = edge[1]
    loop_idx = jnp.where(row == col, row, N)
    has_loop = jnp.zeros((N,), bool).at[loop_idx].set(True, mode='drop')
    loop_w = jnp.where(has_loop, 0.0, 1.0).astype(jnp.float32)
    deg = jnp.zeros((N,), jnp.float32).at[col].add(1.0) + loop_w
    dinv = jnp.where(deg > 0, 1.0 / jnp.sqrt(deg), 0.0)
    all_nodes = jnp.arange(N, dtype=row.dtype)
    row_f = jnp.concatenate([row, all_nodes])
    col_f = jnp.concatenate([col, all_nodes])
    norm_f = jnp.concatenate([dinv[row] * dinv[col], dinv * dinv * loop_w]).astype(jnp.float32)
    return row_f, col_f, norm_f


def _gcn(x, struct, W, b, N):
    row, col, norm = struct
    xw = x @ W
    msg = xw[row] * norm[:, None]
    return jax.ops.segment_sum(msg, col, num_segments=N) + b


def _pair_products_sparse(u, v, row, col, pi, pk, n):
    E = row.shape[0]
    code = row.astype(jnp.int32) * n + col.astype(jnp.int32)
    order = jnp.argsort(code)
    scode = code[order]
    scol = col.astype(jnp.int32)[order]
    us = u[order]
    vs = v[order]
    indptr = jnp.searchsorted(scode, jnp.arange(n + 1, dtype=jnp.int32) * n).astype(jnp.int32)
    deg = indptr[1:] - indptr[:-1]
    lo = indptr[pi]
    hi = indptr[pi + 1]
    P = pi.shape[0]

    def body(t, acc):
        e1 = jnp.minimum(lo + t, E - 1)
        active = (lo + t) < hi
        j = scol[e1]
        target = j * n + pk
        pos = jnp.clip(jnp.searchsorted(scode, target), 0, E - 1)
        found = active & (scode[pos] == target)
        prod = us[e1] * vs[pos] * found[:, None].astype(jnp.float32)
        return acc + prod

    maxdeg = jnp.max(deg)
    acc0 = jnp.zeros((P, u.shape[1]), jnp.float32)
    acc = jax.lax.fori_loop(0, maxdeg, body, acc0)
    return acc


def _head_kernel(h_ref, a4_ref, c4_ref, o_ref):
    o_ref[...] = jax.nn.sigmoid(
        jnp.dot(h_ref[...], a4_ref[...], preferred_element_type=jnp.float32)
        + c4_ref[...]
    )


def _head(h, A4, c4):
    R = h.shape[0]
    BR = 2000
    grid = (R // BR,)
    return pl.pallas_call(
        _head_kernel,
        grid=grid,
        in_specs=[
            pl.BlockSpec((BR, h.shape[1]), lambda i: (i, 0)),
            pl.BlockSpec((h.shape[1], A4.shape[1]), lambda i: (0, 0)),
            pl.BlockSpec((1, A4.shape[1]), lambda i: (0, 0)),
        ],
        out_specs=pl.BlockSpec((BR, A4.shape[1]), lambda i: (i, 0)),
        out_shape=jax.ShapeDtypeStruct((R, A4.shape[1]), jnp.float32),
    )(h, A4, c4.reshape(1, -1))


def kernel(x, W10, b10, W11, b11, Wr0, br0, Wr1, br1, A1, c1, A2, c2, A3, c3, A4, c4, edge2, edge2_r, ei, pred_links, num_node, pos_mask):
    N = x.shape[0]
    n = N_NODE
    nm = jnp.asarray(num_node, dtype=jnp.int32)
    s2 = _gcn_structure(edge2, N)
    s2r = _gcn_structure(edge2_r, N)
    val = x
    h = x
    convs = [(W10, b10), (W11, b11)]
    convs_r = [(Wr0, br0), (Wr1, br1)]
    for i in range(2):
        ha = jax.nn.relu(_ln(_gcn(h, s2, convs[i][0], convs[i][1], N)))
        hb = jax.nn.relu(_ln(_gcn(h, s2r, convs_r[i][0], convs_r[i][1], N)))
        h = ha + hb
    vm = jnp.where(pos_mask[:, None], val, 0.0)
    x1 = jax.nn.relu(_ln(vm @ A1 + c1))
    x2 = jax.nn.relu(_ln(vm @ A2 + c2))
    E = ei.shape[1]
    row = ei[0].astype(jnp.int32)
    col = ei[1].astype(jnp.int32)
    u = jnp.concatenate([x1[:E], jnp.ones((E, 1), jnp.float32)], axis=1)
    v = jnp.concatenate([x2[:E], jnp.ones((E, 1), jnp.float32)], axis=1)
    pi = pred_links[0].astype(jnp.int32)
    pk = pred_links[1].astype(jnp.int32)
    res = _pair_products_sparse(u, v, row, col, pi, pk, n)
    cdim = x1.shape[1]
    cval = res[:, :cdim]
    cnt = res[:, cdim]
    code_ei = row * nm + col
    code_p = pi * nm + pk
    s = jnp.sort(code_ei)
    pos = jnp.clip(jnp.searchsorted(s, code_p), 0, E - 1)
    ind = s[pos] == code_p
    feat = jnp.concatenate([cval, ind.astype(jnp.float32)[:, None]], axis=1)
    value = jax.nn.relu(_ln(feat @ A3 + c3))
    value = jnp.where(((cnt > 0) | ind)[:, None], value, 0.0)
    h = jnp.concatenate([h, value], axis=-1)
    h = h[: h.shape[0] // 2] + h[h.shape[0] // 2 :]
    return _head(h, A4, c4)


# speedup vs baseline: 3.0085x; 1.4584x over previous
import functools

import jax
import jax.numpy as jnp
from jax import lax
from jax.experimental import pallas as pl
from jax.experimental.pallas import tpu as pltpu
from jax.experimental.pallas import tpu_sc as plsc

N_NODE = 10000


def _ln(x, eps=1e-5):
    mu = jnp.mean(x, axis=-1, keepdims=True)
    var = jnp.mean((x - mu) ** 2, axis=-1, keepdims=True)
    return (x - mu) / jnp.sqrt(var + eps)


def _gcn_structure(edge, N):
    row = edge[0]
    col = edge[1]
    loop_idx = jnp.where(row == col, row, N)
    has_loop = jnp.zeros((N,), bool).at[loop_idx].set(True, mode='drop')
    loop_w = jnp.where(has_loop, 0.0, 1.0).astype(jnp.float32)
    deg = jnp.zeros((N,), jnp.float32).at[col].add(1.0) + loop_w
    dinv = jnp.where(deg > 0, 1.0 / jnp.sqrt(deg), 0.0)
    all_nodes = jnp.arange(N, dtype=row.dtype)
    row_f = jnp.concatenate([row, all_nodes])
    col_f = jnp.concatenate([col, all_nodes])
    norm_f = jnp.concatenate([dinv[row] * dinv[col], dinv * dinv * loop_w]).astype(jnp.float32)
    return row_f, col_f, norm_f


def _gcn(x, struct, W, b, N):
    row, col, norm = struct
    xw = x @ W
    msg = xw[row] * norm[:, None]
    return jax.ops.segment_sum(msg, col, num_segments=N) + b


def _pair_products_sparse(u, v, row, col, pi, pk, n):
    E = row.shape[0]
    code = row.astype(jnp.int32) * n + col.astype(jnp.int32)
    order = jnp.argsort(code)
    scode = code[order]
    scol = col.astype(jnp.int32)[order]
    us = u[order]
    vs = v[order]
    indptr = jnp.searchsorted(scode, jnp.arange(n + 1, dtype=jnp.int32) * n).astype(jnp.int32)
    deg = indptr[1:] - indptr[:-1]
    lo = indptr[pi]
    hi = indptr[pi + 1]
    P = pi.shape[0]

    def body(t, acc):
        e1 = jnp.minimum(lo + t, E - 1)
        active = (lo + t) < hi
        j = scol[e1]
        target = j * n + pk
        pos = jnp.clip(jnp.searchsorted(scode, target), 0, E - 1)
        found = active & (scode[pos] == target)
        prod = us[e1] * vs[pos] * found[:, None].astype(jnp.float32)
        return acc + prod

    maxdeg = jnp.max(deg)
    acc0 = jnp.zeros((P, u.shape[1]), jnp.float32)
    acc = jax.lax.fori_loop(0, maxdeg, body, acc0)
    return acc


_NW = 32
_CH = 256
_MB = 1024
_C32 = 32


def _make_pp_kernel(E, P, n):
    assert E % 2 == 0 and P % _CH == 0
    nchunk = P // _CH
    iptr_pad = ((n + 1 + 7) // 8) * 8
    i16 = lambda: lax.iota(jnp.int32, 16)

    def colval(colpk_v, e):
        word = plsc.load_gather(colpk_v, [e >> 1])
        return (word >> ((e & 1) << 4)) & 0xFFFF

    def lower_bound(colpk_v, l0, ln0, key):
        def cond(c):
            return jnp.max(c[1]) > 0

        def body(c):
            lo_, ln_ = c
            half = ln_ >> 1
            mid = lo_ + half
            cm = colval(colpk_v, jnp.minimum(mid, E - 1))
            less = cm < key
            lo2 = jnp.where(less, mid + 1, lo_)
            ln2 = jnp.where(less, ln_ - half - 1, half)
            return (lo2, ln2)

        lo_f, _ = lax.while_loop(cond, body, (l0, ln0))
        return lo_f

    def body(colpk_h, iptr_h, pi_h, pk_h, us_h, vs_h, res_h, ind_h,
             colpk_v, iptr_v, pi_v, pk_v, res_v, ind_v,
             m_pl, m_e1, m_e2, ubuf, vbuf, cnt_s, sem1, sem2):
        wid = lax.axis_index("s") * 2 + lax.axis_index("c")
        pltpu.sync_copy(colpk_h, colpk_v)
        pltpu.sync_copy(iptr_h, iptr_v)
        cnt_s[0] = 0
        zi = jnp.zeros((16,), jnp.int32)

        def init_mb(z, _):
            m_pl[pl.ds(z * 16, 16)] = zi
            m_e1[pl.ds(z * 16, 16)] = zi
            m_e2[pl.ds(z * 16, 16)] = zi
            return 0

        lax.fori_loop(0, _MB // 16, init_mb, 0)

        def flush():
            cnt = cnt_s[0]
            nb = (cnt + 127) // 128

            def fbody(b, _):
                off = b * 128
                d1 = pltpu.async_copy(us_h.at[m_e1.at[pl.ds(off, 128)]], ubuf, sem1)
                d2 = pltpu.async_copy(vs_h.at[m_e2.at[pl.ds(off, 128)]], vbuf, sem2)
                d1.wait()
                d2.wait()
                for g in range(8):
                    valid = (off + g * 16 + i16()) < cnt
                    plv = m_pl[pl.ds(off + g * 16, 16)]
                    midx = g * 16 + i16()

                    def cbody(c, _):
                        cs = jnp.zeros((16,), jnp.int32) + c
                        uv = plsc.load_gather(ubuf, [midx, cs])
                        vv = plsc.load_gather(vbuf, [midx, cs])
                        plsc.addupdate_scatter(res_v, [plv * _C32 + c], uv * vv,
                                               mask=valid)
                        return 0

                    lax.fori_loop(0, _C32, cbody, 0)
                return 0

            lax.fori_loop(0, nb, fbody, 0)
            cnt_s[0] = 0

        def chunk_body(it, _):
            cbase = (wid + it * _NW) * _CH
            pltpu.sync_copy(pi_h.at[pl.ds(cbase, _CH)], pi_v)
            pltpu.sync_copy(pk_h.at[pl.ds(cbase, _CH)], pk_v)

            def zbody(z, _):
                res_v[pl.ds(z * 16, 16)] = jnp.zeros((16,), jnp.float32)
                return 0

            lax.fori_loop(0, _CH * _C32 // 16, zbody, 0)

            def gloop(grp, _):
                lb = grp * 16
                vi = pi_v[pl.ds(lb, 16)]
                vk = pk_v[pl.ds(lb, 16)]
                lo = plsc.load_gather(iptr_v, [vi])
                hi = plsc.load_gather(iptr_v, [vi + 1])
                deg = hi - lo
                pos = lower_bound(colpk_v, lo, deg, vk)
                fnd = ((pos < hi)
                       & (colval(colpk_v, jnp.minimum(pos, E - 1)) == vk))
                ind_v[pl.ds(lb, 16)] = jnp.where(fnd, 1.0, 0.0)
                dmax = jnp.max(deg)

                def tbody(t, _):
                    @pl.when(cnt_s[0] > _MB - 16)
                    def _():
                        flush()

                    e1 = jnp.minimum(lo + t, E - 1)
                    act = t < deg
                    j = colval(colpk_v, e1)
                    jlo = plsc.load_gather(iptr_v, [j])
                    jhi = plsc.load_gather(iptr_v, [j + 1])
                    ln0 = jnp.where(act, jhi - jlo, 0)
                    pos2 = lower_bound(colpk_v, jlo, ln0, vk)
                    e2 = jnp.minimum(pos2, E - 1)
                    fd = act & (pos2 < jhi) & (colval(colpk_v, e2) == vk)
                    cnt = cnt_s[0]
                    plsc.store_compressed(m_pl.at[pl.ds(cnt, 16)], lb + i16(),
                                          mask=fd)
                    plsc.store_compressed(m_e1.at[pl.ds(cnt, 16)], e1, mask=fd)
                    plsc.store_compressed(m_e2.at[pl.ds(cnt, 16)], e2, mask=fd)
                    cnt_s[0] = cnt + jnp.sum(jnp.where(fd, 1, 0))
                    return 0

                lax.fori_loop(0, dmax, tbody, 0)
                return 0

            lax.fori_loop(0, _CH // 16, gloop, 0)
            flush()
            pltpu.sync_copy(res_v, res_h.at[pl.ds(cbase * _C32, _CH * _C32)])
            pltpu.sync_copy(ind_v, ind_h.at[pl.ds(cbase, _CH)])
            return 0

        niter = (nchunk - wid + _NW - 1) // _NW
        lax.fori_loop(0, niter, chunk_body, 0)

    mesh = plsc.VectorSubcoreMesh(core_axis_name="c", subcore_axis_name="s")
    return pl.kernel(
        body,
        out_type=[
            jax.ShapeDtypeStruct((P * _C32,), jnp.float32),
            jax.ShapeDtypeStruct((P,), jnp.float32),
        ],
        mesh=mesh,
        compiler_params=pltpu.CompilerParams(needs_layout_passes=False,
                                             use_tc_tiling_on_sc=False),
        scratch_types=[
            pltpu.VMEM((E // 2,), jnp.int32),
            pltpu.VMEM((iptr_pad,), jnp.int32),
            pltpu.VMEM((_CH,), jnp.int32),
            pltpu.VMEM((_CH,), jnp.int32),
            pltpu.VMEM((_CH * _C32,), jnp.float32),
            pltpu.VMEM((_CH,), jnp.float32),
            pltpu.VMEM((_MB,), jnp.int32),
            pltpu.VMEM((_MB,), jnp.int32),
            pltpu.VMEM((_MB,), jnp.int32),
            pltpu.VMEM((128, _C32), jnp.float32),
            pltpu.VMEM((128, _C32), jnp.float32),
            pltpu.SMEM((8,), jnp.int32),
            pltpu.SemaphoreType.DMA,
            pltpu.SemaphoreType.DMA,
        ],
    )


def _pair_products_sc(u, v, row, col, pi, pk, n):
    E = row.shape[0]
    P = pi.shape[0]
    code = row * n + col
    order = jnp.argsort(code)
    scode = code[order]
    scol = col[order]
    us = jnp.zeros((E, _C32), jnp.float32).at[:, : u.shape[1]].set(u[order])
    vs = jnp.zeros((E, _C32), jnp.float32).at[:, : v.shape[1]].set(v[order])
    colpk = scol[0::2] | (scol[1::2] << 16)
    iptr_pad = ((n + 1 + 7) // 8) * 8
    iptr = jnp.full((iptr_pad,), E, jnp.int32).at[: n + 1].set(
        jnp.searchsorted(scode, jnp.arange(n + 1, dtype=jnp.int32) * n)
        .astype(jnp.int32))
    res_flat, ind = _make_pp_kernel(E, P, n)(colpk, iptr, pi, pk, us, vs)
    return res_flat.reshape(P, _C32), ind


def _head_kernel(h_ref, a4_ref, c4_ref, o_ref):
    o_ref[...] = jax.nn.sigmoid(
        jnp.dot(h_ref[...], a4_ref[...], preferred_element_type=jnp.float32)
        + c4_ref[...]
    )


def _head(h, A4, c4):
    R = h.shape[0]
    BR = 2000
    grid = (R // BR,)
    return pl.pallas_call(
        _head_kernel,
        grid=grid,
        in_specs=[
            pl.BlockSpec((BR, h.shape[1]), lambda i: (i, 0)),
            pl.BlockSpec((h.shape[1], A4.shape[1]), lambda i: (0, 0)),
            pl.BlockSpec((1, A4.shape[1]), lambda i: (0, 0)),
        ],
        out_specs=pl.BlockSpec((BR, A4.shape[1]), lambda i: (i, 0)),
        out_shape=jax.ShapeDtypeStruct((R, A4.shape[1]), jnp.float32),
    )(h, A4, c4.reshape(1, -1))


def kernel(x, W10, b10, W11, b11, Wr0, br0, Wr1, br1, A1, c1, A2, c2, A3, c3, A4, c4, edge2, edge2_r, ei, pred_links, num_node, pos_mask):
    N = x.shape[0]
    n = N_NODE
    nm = jnp.asarray(num_node, dtype=jnp.int32)
    s2 = _gcn_structure(edge2, N)
    s2r = _gcn_structure(edge2_r, N)
    val = x
    h = x
    convs = [(W10, b10), (W11, b11)]
    convs_r = [(Wr0, br0), (Wr1, br1)]
    for i in range(2):
        ha = jax.nn.relu(_ln(_gcn(h, s2, convs[i][0], convs[i][1], N)))
        hb = jax.nn.relu(_ln(_gcn(h, s2r, convs_r[i][0], convs_r[i][1], N)))
        h = ha + hb
    vm = jnp.where(pos_mask[:, None], val, 0.0)
    x1 = jax.nn.relu(_ln(vm @ A1 + c1))
    x2 = jax.nn.relu(_ln(vm @ A2 + c2))
    E = ei.shape[1]
    row = ei[0].astype(jnp.int32)
    col = ei[1].astype(jnp.int32)
    u = jnp.concatenate([x1[:E], jnp.ones((E, 1), jnp.float32)], axis=1)
    v = jnp.concatenate([x2[:E], jnp.ones((E, 1), jnp.float32)], axis=1)
    pi = pred_links[0].astype(jnp.int32)
    pk = pred_links[1].astype(jnp.int32)
    res, ind_f = _pair_products_sc(u, v, row, col, pi, pk, n)
    cdim = x1.shape[1]
    cval = res[:, :cdim]
    cnt = res[:, cdim]
    ind = ind_f > 0.5
    feat = jnp.concatenate([cval, ind_f[:, None]], axis=1)
    value = jax.nn.relu(_ln(feat @ A3 + c3))
    value = jnp.where(((cnt > 0) | ind)[:, None], value, 0.0)
    h = jnp.concatenate([h, value], axis=-1)
    h = h[: h.shape[0] // 2] + h[h.shape[0] // 2 :]
    return _head(h, A4, c4)


# speedup vs baseline: 46.7207x; 15.5298x over previous
import functools

import jax
import jax.numpy as jnp
from jax import lax
from jax.experimental import pallas as pl
from jax.experimental.pallas import tpu as pltpu
from jax.experimental.pallas import tpu_sc as plsc

N_NODE = 10000


def _ln(x, eps=1e-5):
    mu = jnp.mean(x, axis=-1, keepdims=True)
    var = jnp.mean((x - mu) ** 2, axis=-1, keepdims=True)
    return (x - mu) / jnp.sqrt(var + eps)


def _gcn_structure(edge, N):
    row = edge[0]
    col = edge[1]
    loop_idx = jnp.where(row == col, row, N)
    has_loop = jnp.zeros((N,), bool).at[loop_idx].set(True, mode='drop')
    loop_w = jnp.where(has_loop, 0.0, 1.0).astype(jnp.float32)
    deg = jnp.zeros((N,), jnp.float32).at[col].add(1.0) + loop_w
    dinv = jnp.where(deg > 0, 1.0 / jnp.sqrt(deg), 0.0)
    all_nodes = jnp.arange(N, dtype=row.dtype)
    row_f = jnp.concatenate([row, all_nodes])
    col_f = jnp.concatenate([col, all_nodes])
    norm_f = jnp.concatenate([dinv[row] * dinv[col], dinv * dinv * loop_w]).astype(jnp.float32)
    return row_f, col_f, norm_f


def _gcn(x, struct, W, b, N):
    row, col, norm = struct
    xw = x @ W
    msg = xw[row] * norm[:, None]
    return jax.ops.segment_sum(msg, col, num_segments=N) + b


def _pair_products_sparse(u, v, row, col, pi, pk, n):
    E = row.shape[0]
    code = row.astype(jnp.int32) * n + col.astype(jnp.int32)
    order = jnp.argsort(code)
    scode = code[order]
    scol = col.astype(jnp.int32)[order]
    us = u[order]
    vs = v[order]
    indptr = jnp.searchsorted(scode, jnp.arange(n + 1, dtype=jnp.int32) * n).astype(jnp.int32)
    deg = indptr[1:] - indptr[:-1]
    lo = indptr[pi]
    hi = indptr[pi + 1]
    P = pi.shape[0]

    def body(t, acc):
        e1 = jnp.minimum(lo + t, E - 1)
        active = (lo + t) < hi
        j = scol[e1]
        target = j * n + pk
        pos = jnp.clip(jnp.searchsorted(scode, target), 0, E - 1)
        found = active & (scode[pos] == target)
        prod = us[e1] * vs[pos] * found[:, None].astype(jnp.float32)
        return acc + prod

    maxdeg = jnp.max(deg)
    acc0 = jnp.zeros((P, u.shape[1]), jnp.float32)
    acc = jax.lax.fori_loop(0, maxdeg, body, acc0)
    return acc


_NW = 32
_CH = 256
_MB = 1024
_C32 = 32


def _make_pp_kernel(E, P, n):
    assert E % 2 == 0 and P % _CH == 0
    nchunk = P // _CH
    iptr_pad = ((n + 1 + 7) // 8) * 8
    i16 = lambda: lax.iota(jnp.int32, 16)

    def colval(colpk_v, e):
        word = plsc.load_gather(colpk_v, [e >> 1])
        return (word >> ((e & 1) << 4)) & 0xFFFF

    def lower_bound(colpk_v, l0, ln0, key):
        def cond(c):
            return jnp.max(c[1]) > 0

        def body(c):
            lo_, ln_ = c
            half = ln_ >> 1
            mid = lo_ + half
            cm = colval(colpk_v, jnp.minimum(mid, E - 1))
            less = cm < key
            lo2 = jnp.where(less, mid + 1, lo_)
            ln2 = jnp.where(less, ln_ - half - 1, half)
            return (lo2, ln2)

        lo_f, _ = lax.while_loop(cond, body, (l0, ln0))
        return lo_f

    def body(colpk_h, iptr_h, pi_h, pk_h, us_h, vs_h, res_h, ind_h,
             colpk_v, iptr_v, pi_v, pk_v, res_v, ind_v,
             m_pl, m_e1, m_e2, ubuf, vbuf, cnt_s, sem1, sem2):
        wid = lax.axis_index("s") * 2 + lax.axis_index("c")
        pltpu.sync_copy(colpk_h, colpk_v)
        pltpu.sync_copy(iptr_h, iptr_v)
        cnt_s[0] = 0
        zi = jnp.zeros((16,), jnp.int32)

        def init_mb(z, _):
            m_pl[pl.ds(z * 16, 16)] = zi
            m_e1[pl.ds(z * 16, 16)] = zi
            m_e2[pl.ds(z * 16, 16)] = zi
            return 0

        lax.fori_loop(0, _MB // 16, init_mb, 0)

        def flush():
            cnt = cnt_s[0]
            nb = (cnt + 127) // 128

            def fbody(b, _):
                off = b * 128
                d1 = pltpu.async_copy(us_h.at[m_e1.at[pl.ds(off, 128)]], ubuf, sem1)
                d2 = pltpu.async_copy(vs_h.at[m_e2.at[pl.ds(off, 128)]], vbuf, sem2)
                d1.wait()
                d2.wait()
                for g in range(8):
                    valid = (off + g * 16 + i16()) < cnt
                    plv = m_pl[pl.ds(off + g * 16, 16)]
                    midx = g * 16 + i16()

                    def cbody(c, _):
                        cs = jnp.zeros((16,), jnp.int32) + c
                        uv = plsc.load_gather(ubuf, [midx, cs])
                        vv = plsc.load_gather(vbuf, [midx, cs])
                        plsc.addupdate_scatter(res_v, [plv * _C32 + c], uv * vv,
                                               mask=valid)
                        return 0

                    lax.fori_loop(0, _C32, cbody, 0)
                return 0

            lax.fori_loop(0, nb, fbody, 0)
            cnt_s[0] = 0

        def chunk_body(it, _):
            cbase = (wid + it * _NW) * _CH
            pltpu.sync_copy(pi_h.at[pl.ds(cbase, _CH)], pi_v)
            pltpu.sync_copy(pk_h.at[pl.ds(cbase, _CH)], pk_v)

            def zbody(z, _):
                res_v[pl.ds(z * 16, 16)] = jnp.zeros((16,), jnp.float32)
                return 0

            lax.fori_loop(0, _CH * _C32 // 16, zbody, 0)

            def gloop(grp, _):
                lb = grp * 16
                vi = pi_v[pl.ds(lb, 16)]
                vk = pk_v[pl.ds(lb, 16)]
                lo = plsc.load_gather(iptr_v, [vi])
                hi = plsc.load_gather(iptr_v, [vi + 1])
                deg = hi - lo
                pos = lower_bound(colpk_v, lo, deg, vk)
                fnd = ((pos < hi)
                       & (colval(colpk_v, jnp.minimum(pos, E - 1)) == vk))
                ind_v[pl.ds(lb, 16)] = jnp.where(fnd, 1.0, 0.0)
                dmax = jnp.max(deg)

                def tbody(t, _):
                    @pl.when(cnt_s[0] > _MB - 16)
                    def _():
                        flush()

                    e1 = jnp.minimum(lo + t, E - 1)
                    act = t < deg
                    j = colval(colpk_v, e1)
                    jlo = plsc.load_gather(iptr_v, [j])
                    jhi = plsc.load_gather(iptr_v, [j + 1])
                    ln0 = jnp.where(act, jhi - jlo, 0)
                    pos2 = lower_bound(colpk_v, jlo, ln0, vk)
                    e2 = jnp.minimum(pos2, E - 1)
                    fd = act & (pos2 < jhi) & (colval(colpk_v, e2) == vk)
                    cnt = cnt_s[0]
                    plsc.store_compressed(m_pl.at[pl.ds(cnt, 16)], lb + i16(),
                                          mask=fd)
                    plsc.store_compressed(m_e1.at[pl.ds(cnt, 16)], e1, mask=fd)
                    plsc.store_compressed(m_e2.at[pl.ds(cnt, 16)], e2, mask=fd)
                    cnt_s[0] = cnt + jnp.sum(jnp.where(fd, 1, 0))
                    return 0

                lax.fori_loop(0, dmax, tbody, 0)
                return 0

            lax.fori_loop(0, _CH // 16, gloop, 0)
            flush()
            pltpu.sync_copy(res_v, res_h.at[pl.ds(cbase * _C32, _CH * _C32)])
            pltpu.sync_copy(ind_v, ind_h.at[pl.ds(cbase, _CH)])
            return 0

        niter = (nchunk - wid + _NW - 1) // _NW
        lax.fori_loop(0, niter, chunk_body, 0)

    mesh = plsc.VectorSubcoreMesh(core_axis_name="c", subcore_axis_name="s")
    return pl.kernel(
        body,
        out_type=[
            jax.ShapeDtypeStruct((P * _C32,), jnp.float32),
            jax.ShapeDtypeStruct((P,), jnp.float32),
        ],
        mesh=mesh,
        compiler_params=pltpu.CompilerParams(needs_layout_passes=False,
                                             use_tc_tiling_on_sc=False),
        scratch_types=[
            pltpu.VMEM((E // 2,), jnp.int32),
            pltpu.VMEM((iptr_pad,), jnp.int32),
            pltpu.VMEM((_CH,), jnp.int32),
            pltpu.VMEM((_CH,), jnp.int32),
            pltpu.VMEM((_CH * _C32,), jnp.float32),
            pltpu.VMEM((_CH,), jnp.float32),
            pltpu.VMEM((_MB,), jnp.int32),
            pltpu.VMEM((_MB,), jnp.int32),
            pltpu.VMEM((_MB,), jnp.int32),
            pltpu.VMEM((128, _C32), jnp.float32),
            pltpu.VMEM((128, _C32), jnp.float32),
            pltpu.SMEM((8,), jnp.int32),
            pltpu.SemaphoreType.DMA,
            pltpu.SemaphoreType.DMA,
        ],
    )


def _pair_products_sc(u, v, row, col, pi, pk, n):
    E = row.shape[0]
    P = pi.shape[0]
    code = row * n + col
    order = jnp.argsort(code)
    scode = code[order]
    scol = col[order]
    us = jnp.zeros((E, _C32), jnp.float32).at[:, : u.shape[1]].set(u[order])
    vs = jnp.zeros((E, _C32), jnp.float32).at[:, : v.shape[1]].set(v[order])
    colpk = scol[0::2] | (scol[1::2] << 16)
    iptr_pad = ((n + 1 + 7) // 8) * 8
    iptr = jnp.full((iptr_pad,), E, jnp.int32).at[: n + 1].set(
        jnp.searchsorted(scode, jnp.arange(n + 1, dtype=jnp.int32) * n)
        .astype(jnp.int32))
    res_flat, ind = _make_pp_kernel(E, P, n)(colpk, iptr, pi, pk, us, vs)
    return res_flat.reshape(P, _C32), ind


_M2 = 2560000
_ET = _M2 // _NW
_B1 = 2000
_NGB = _B1 // 16
_NBK = _ET // _B1
_CN = 53504
_CNA = 53760
_NCHK = 3
_PEND = 512
_FLUSH = 384
_NS_ROWS = 160000
_NSA = 160256


def _make_gcn_msg_kernel():
    rows_pt = _CNA // 16
    wb_pt = _CN // 16
    dump = _CN

    def body(row_h, col_h, y_h, zeros_h, acc_h,
             row_v, col_v, pend_r, pend_c, ybuf, acc_sp, ps, sem1, sem2):
        cid = lax.axis_index("c")
        sid = lax.axis_index("s")
        wid = sid * 2 + cid
        i16 = lax.iota(jnp.int32, 16)
        eb = wid * _ET

        def init_pend(z, _):
            pend_r[pl.ds(z * 16, 16)] = jnp.zeros((16,), jnp.int32)
            pend_c[pl.ds(z * 16, 16)] = jnp.zeros((16,), jnp.int32) + dump
            return 0

        lax.fori_loop(0, _PEND // 16, init_pend, 0)

        for ch in range(_NCHK):
            base = ch * _CN
            pltpu.sync_copy(zeros_h, acc_sp.at[pl.ds(sid * rows_pt, rows_pt), :])
            plsc.subcore_barrier()
            ps[0] = 0

            def flush3():
                ds_g = []
                for b in range(3):
                    ds_g.append(pltpu.async_copy(
                        y_h.at[pend_r.at[pl.ds(b * 128, 128)]],
                        ybuf.at[pl.ds(b * 128, 128), :], sem1))
                for d in ds_g:
                    d.wait()
                ds_s = []
                for b in range(3):
                    ds_s.append(pltpu.async_copy(
                        ybuf.at[pl.ds(b * 128, 128), :],
                        acc_sp.at[pend_c.at[pl.ds(b * 128, 128)]], sem2,
                        add=True))
                for d in ds_s:
                    d.wait()
                pend_r[pl.ds(0, 16)] = pend_r[pl.ds(_FLUSH, 16)]
                pend_c[pl.ds(0, 16)] = pend_c[pl.ds(_FLUSH, 16)]
                ps[0] = ps[0] - _FLUSH

            def blk(bk, _):
                off = eb + bk * _B1
                pltpu.sync_copy(row_h.at[pl.ds(off, _B1)], row_v)
                pltpu.sync_copy(col_h.at[pl.ds(off, _B1)], col_v)

                def grp(g, _):
                    r16 = row_v[pl.ds(g * 16, 16)]
                    c16 = col_v[pl.ds(g * 16, 16)]
                    m = (c16 >= base) & (c16 < base + _CN)
                    pcnt = ps[0]
                    plsc.store_compressed(pend_r.at[pl.ds(pcnt, 16)], r16,
                                          mask=m)
                    plsc.store_compressed(pend_c.at[pl.ds(pcnt, 16)],
                                          c16 - base, mask=m)
                    ps[0] = pcnt + jnp.sum(jnp.where(m, 1, 0))

                    @pl.when(ps[0] >= _FLUSH)
                    def _():
                        flush3()

                    return 0

                lax.fori_loop(0, _NGB, grp, 0)
                return 0

            lax.fori_loop(0, _NBK, blk, 0)

            pcnt = ps[0]
            for g in range(_FLUSH // 16):
                idx = g * 16 + i16
                plsc.store_scatter(pend_c, [idx],
                                   jnp.zeros((16,), jnp.int32) + dump,
                                   mask=idx >= pcnt)
            nb = (pcnt + 127) // 128

            def dr(b, _):
                pltpu.async_copy(
                    y_h.at[pend_r.at[pl.ds(b * 128, 128)]],
                    ybuf.at[pl.ds(b * 128, 128), :], sem1).wait()
                pltpu.sync_copy(ybuf.at[pl.ds(b * 128, 128), :],
                                acc_sp.at[pend_c.at[pl.ds(b * 128, 128)]],
                                add=True)
                return 0

            lax.fori_loop(0, nb, dr, 0)
            ps[0] = 0
            plsc.subcore_barrier()
            pltpu.sync_copy(
                acc_sp.at[pl.ds(sid * wb_pt, wb_pt), :],
                acc_h.at[cid, pl.ds(base + sid * wb_pt, wb_pt), :])
            plsc.subcore_barrier()

    mesh = plsc.VectorSubcoreMesh(core_axis_name="c", subcore_axis_name="s")
    return pl.kernel(
        body,
        out_type=[jax.ShapeDtypeStruct((2, _NCHK * _CN, _C32), jnp.float32)],
        mesh=mesh,
        compiler_params=pltpu.CompilerParams(needs_layout_passes=False,
                                             use_tc_tiling_on_sc=False),
        scratch_types=[
            pltpu.VMEM((_B1,), jnp.int32),
            pltpu.VMEM((_B1,), jnp.int32),
            pltpu.VMEM((_PEND,), jnp.int32),
            pltpu.VMEM((_PEND,), jnp.int32),
            pltpu.VMEM((_FLUSH, _C32), jnp.float32),
            pltpu.VMEM_SHARED((_CNA, _C32), jnp.float32),
            pltpu.SMEM((8,), jnp.int32),
            pltpu.SemaphoreType.DMA,
            pltpu.SemaphoreType.DMA,
        ],
    )


def _make_gcn_struct_kernel():
    rows_pt = _NSA // 16
    wb_pt = _NS_ROWS // 16
    dump = _NS_ROWS

    def body(row_h, col_h, zeros_h, acc_h,
             row_v, col_v, pend_c, pend_il, pay, acc_sp, ps, sem2):
        cid = lax.axis_index("c")
        sid = lax.axis_index("s")
        wid = sid * 2 + cid
        i16 = lax.iota(jnp.int32, 16)
        eb = wid * _ET
        tru = jnp.ones((16,), jnp.bool_)

        def init_pend(z, _):
            pend_c[pl.ds(z * 16, 16)] = jnp.zeros((16,), jnp.int32) + dump
            pend_il[pl.ds(z * 16, 16)] = jnp.zeros((16,), jnp.float32)
            return 0

        lax.fori_loop(0, _PEND // 16, init_pend, 0)
        zrow = jnp.zeros((16,), jnp.float32)
        pltpu.sync_copy(zeros_h.at[pl.ds(0, _FLUSH), :], pay)

        def set_ones(g, _):
            plsc.store_scatter(pay, [g * 16 + i16, i16 * 0], zrow + 1.0,
                               mask=tru)
            return 0

        lax.fori_loop(0, _FLUSH // 16, set_ones, 0)

        pltpu.sync_copy(zeros_h, acc_sp.at[pl.ds(sid * rows_pt, rows_pt), :])
        plsc.subcore_barrier()
        ps[0] = 0

        def expand_il(nslots):
            def ex(g, _):
                il = pend_il[pl.ds(g * 16, 16)]
                plsc.store_scatter(pay, [g * 16 + i16, i16 * 0 + 1], il,
                                   mask=tru)
                return 0

            lax.fori_loop(0, nslots // 16, ex, 0)

        def flush3():
            expand_il(_FLUSH)
            ds_s = []
            for b in range(3):
                ds_s.append(pltpu.async_copy(
                    pay.at[pl.ds(b * 128, 128), :],
                    acc_sp.at[pend_c.at[pl.ds(b * 128, 128)]], sem2,
                    add=True))
            for d in ds_s:
                d.wait()
            pend_c[pl.ds(0, 16)] = pend_c[pl.ds(_FLUSH, 16)]
            pend_il[pl.ds(0, 16)] = pend_il[pl.ds(_FLUSH, 16)]
            ps[0] = ps[0] - _FLUSH

        def blk(bk, _):
            off = eb + bk * _B1
            pltpu.sync_copy(row_h.at[pl.ds(off, _B1)], row_v)
            pltpu.sync_copy(col_h.at[pl.ds(off, _B1)], col_v)

            def grp(g, _):
                r16 = row_v[pl.ds(g * 16, 16)]
                c16 = col_v[pl.ds(g * 16, 16)]
                il = jnp.where(r16 == c16, 1.0, 0.0)
                pcnt = ps[0]
                plsc.store_compressed(pend_c.at[pl.ds(pcnt, 16)], c16,
                                      mask=tru)
                plsc.store_compressed(pend_il.at[pl.ds(pcnt, 16)], il,
                                      mask=tru)
                ps[0] = pcnt + 16

                @pl.when(ps[0] >= _FLUSH)
                def _():
                    flush3()

                return 0

            lax.fori_loop(0, _NGB, grp, 0)
            return 0

        lax.fori_loop(0, _NBK, blk, 0)

        pcnt = ps[0]
        for g in range(_FLUSH // 16):
            idx = g * 16 + i16
            plsc.store_scatter(pend_c, [idx],
                               jnp.zeros((16,), jnp.int32) + dump,
                               mask=idx >= pcnt)
            plsc.store_scatter(pend_il, [idx], jnp.zeros((16,), jnp.float32),
                               mask=idx >= pcnt)
        expand_il(_FLUSH)
        nb = (pcnt + 127) // 128

        def dr(b, _):
            pltpu.sync_copy(pay.at[pl.ds(b * 128, 128), :],
                            acc_sp.at[pend_c.at[pl.ds(b * 128, 128)]],
                            add=True)
            return 0

        lax.fori_loop(0, nb, dr, 0)
        plsc.subcore_barrier()
        pltpu.sync_copy(
            acc_sp.at[pl.ds(sid * wb_pt, wb_pt), :],
            acc_h.at[cid, pl.ds(sid * wb_pt, wb_pt), :])

    mesh = plsc.VectorSubcoreMesh(core_axis_name="c", subcore_axis_name="s")
    return pl.kernel(
        body,
        out_type=[jax.ShapeDtypeStruct((2, _NS_ROWS, 8), jnp.float32)],
        mesh=mesh,
        compiler_params=pltpu.CompilerParams(needs_layout_passes=False,
                                             use_tc_tiling_on_sc=False),
        scratch_types=[
            pltpu.VMEM((_B1,), jnp.int32),
            pltpu.VMEM((_B1,), jnp.int32),
            pltpu.VMEM((_PEND,), jnp.int32),
            pltpu.VMEM((_PEND,), jnp.float32),
            pltpu.VMEM((_FLUSH, 8), jnp.float32),
            pltpu.VMEM_SHARED((_NSA, 8), jnp.float32),
            pltpu.SMEM((8,), jnp.int32),
            pltpu.SemaphoreType.DMA,
        ],
    )


_BR = 2000


def _ln_rows(x, eps=1e-5):
    mu = jnp.mean(x, axis=-1, keepdims=True)
    var = jnp.mean((x - mu) ** 2, axis=-1, keepdims=True)
    return (x - mu) / jnp.sqrt(var + eps)


def _ymat_kernel(h_ref, w_ref, dinv_ref, o_ref):
    o_ref[...] = dinv_ref[...] * jnp.dot(
        h_ref[...], w_ref[...], preferred_element_type=jnp.float32)


def _ymat(h, W, dinv):
    N, K = h.shape
    F = W.shape[1]
    return pl.pallas_call(
        _ymat_kernel,
        grid=(N // _BR,),
        in_specs=[
            pl.BlockSpec((_BR, K), lambda i: (i, 0)),
            pl.BlockSpec((K, F), lambda i: (0, 0)),
            pl.BlockSpec((_BR, 1), lambda i: (i, 0)),
        ],
        out_specs=pl.BlockSpec((_BR, F), lambda i: (i, 0)),
        out_shape=jax.ShapeDtypeStruct((N, F), jnp.float32),
    )(h, W, dinv)


def _comb_kernel(acc_ref, y_ref, dinv_ref, lw_ref, b_ref, o_ref):
    a = acc_ref[0] + acc_ref[1]
    o = dinv_ref[...] * (a + lw_ref[...] * y_ref[...]) + b_ref[...]
    o_ref[...] = jax.nn.relu(_ln_rows(o))


def _comb(acc, y, dinv, lw, b):
    N, F = y.shape
    return pl.pallas_call(
        _comb_kernel,
        grid=(N // _BR,),
        in_specs=[
            pl.BlockSpec((2, _BR, F), lambda i: (0, i, 0)),
            pl.BlockSpec((_BR, F), lambda i: (i, 0)),
            pl.BlockSpec((_BR, 1), lambda i: (i, 0)),
            pl.BlockSpec((_BR, 1), lambda i: (i, 0)),
            pl.BlockSpec((1, F), lambda i: (0, 0)),
        ],
        out_specs=pl.BlockSpec((_BR, F), lambda i: (i, 0)),
        out_shape=jax.ShapeDtypeStruct((N, F), jnp.float32),
    )(acc, y, dinv, lw, b.reshape(1, -1))


def _dinv_kernel(acc_ref, dinv_ref, lw_ref):
    cnt = acc_ref[0, :, 0] + acc_ref[1, :, 0]
    il = acc_ref[0, :, 1] + acc_ref[1, :, 1]
    lw = jnp.where(il > 0, 0.0, 1.0)
    deg = cnt + lw
    dinv_ref[...] = jnp.where(deg > 0, jax.lax.rsqrt(jnp.maximum(deg, 1e-30)),
                              0.0)[:, None]
    lw_ref[...] = lw[:, None]


def _dinv(acc):
    N = acc.shape[1]
    return pl.pallas_call(
        _dinv_kernel,
        grid=(N // _BR,),
        in_specs=[pl.BlockSpec((2, _BR, 8), lambda i: (0, i, 0))],
        out_specs=[
            pl.BlockSpec((_BR, 1), lambda i: (i, 0)),
            pl.BlockSpec((_BR, 1), lambda i: (i, 0)),
        ],
        out_shape=[
            jax.ShapeDtypeStruct((N, 1), jnp.float32),
            jax.ShapeDtypeStruct((N, 1), jnp.float32),
        ],
    )(acc)


def _proj_kernel(x_ref, a_ref, c_ref, o_ref):
    o_ref[...] = jax.nn.relu(_ln_rows(
        jnp.dot(x_ref[...], a_ref[...], preferred_element_type=jnp.float32)
        + c_ref[...]))


def _proj(x, A, c):
    N, K = x.shape
    F = A.shape[1]
    return pl.pallas_call(
        _proj_kernel,
        grid=(N // _BR,),
        in_specs=[
            pl.BlockSpec((_BR, K), lambda i: (i, 0)),
            pl.BlockSpec((K, F), lambda i: (0, 0)),
            pl.BlockSpec((1, F), lambda i: (0, 0)),
        ],
        out_specs=pl.BlockSpec((_BR, F), lambda i: (i, 0)),
        out_shape=jax.ShapeDtypeStruct((N, F), jnp.float32),
    )(x, A, c.reshape(1, -1))


def _value_kernel(res_ref, ind_ref, a3p_ref, a3i_ref, c3_ref, o_ref):
    res = res_ref[...]
    ind = ind_ref[...]
    o = (jnp.dot(res, a3p_ref[...], preferred_element_type=jnp.float32)
         + ind * a3i_ref[...] + c3_ref[...])
    o = jax.nn.relu(_ln_rows(o))
    keep = (res[:, 24:25] > 0) | (ind > 0.5)
    o_ref[...] = jnp.where(keep, o, 0.0)


def _value_head(res, ind, A3, c3):
    P = res.shape[0]
    F = A3.shape[1]
    a3p = jnp.zeros((_C32, F), jnp.float32).at[:24].set(A3[:24])
    a3i = A3[24].reshape(1, F)
    return pl.pallas_call(
        _value_kernel,
        grid=(P // _BR,),
        in_specs=[
            pl.BlockSpec((_BR, _C32), lambda i: (i, 0)),
            pl.BlockSpec((_BR, 1), lambda i: (i, 0)),
            pl.BlockSpec((_C32, F), lambda i: (0, 0)),
            pl.BlockSpec((1, F), lambda i: (0, 0)),
            pl.BlockSpec((1, F), lambda i: (0, 0)),
        ],
        out_specs=pl.BlockSpec((_BR, F), lambda i: (i, 0)),
        out_shape=jax.ShapeDtypeStruct((P, F), jnp.float32),
    )(res, ind.reshape(P, 1), a3p, a3i, c3.reshape(1, -1))


def _fold_head_kernel(ht_ref, hb_ref, vt_ref, vb_ref, a4_ref, c4_ref, o_ref):
    h = jnp.concatenate([ht_ref[...] + hb_ref[...],
                         vt_ref[...] + vb_ref[...]], axis=1)
    o_ref[...] = jax.nn.sigmoid(
        jnp.dot(h, a4_ref[...], preferred_element_type=jnp.float32)
        + c4_ref[...])


def _fold_head(hg, value, A4, c4):
    N, F1 = hg.shape
    F2 = value.shape[1]
    R = N // 2
    G = R // _BR
    return pl.pallas_call(
        _fold_head_kernel,
        grid=(G,),
        in_specs=[
            pl.BlockSpec((_BR, F1), lambda i: (i, 0)),
            pl.BlockSpec((_BR, F1), lambda i: (i + G, 0)),
            pl.BlockSpec((_BR, F2), lambda i: (i, 0)),
            pl.BlockSpec((_BR, F2), lambda i: (i + G, 0)),
            pl.BlockSpec((F1 + F2, A4.shape[1]), lambda i: (0, 0)),
            pl.BlockSpec((1, A4.shape[1]), lambda i: (0, 0)),
        ],
        out_specs=pl.BlockSpec((_BR, A4.shape[1]), lambda i: (i, 0)),
        out_shape=jax.ShapeDtypeStruct((R, A4.shape[1]), jnp.float32),
    )(hg, hg, value, value, A4, c4.reshape(1, -1))


def kernel(x, W10, b10, W11, b11, Wr0, br0, Wr1, br1, A1, c1, A2, c2, A3, c3, A4, c4, edge2, edge2_r, ei, pred_links, num_node, pos_mask):
    n = N_NODE
    row2 = edge2[0].astype(jnp.int32)
    col2 = edge2[1].astype(jnp.int32)
    row2r = edge2_r[0].astype(jnp.int32)
    col2r = edge2_r[1].astype(jnp.int32)

    skern = _make_gcn_struct_kernel()
    zeros8 = jnp.zeros((_NSA // 16, 8), jnp.float32)
    (acc_sa,) = skern(row2, col2, zeros8)
    (acc_sb,) = skern(row2r, col2r, zeros8)
    dinv_a, lw_a = _dinv(acc_sa)
    dinv_b, lw_b = _dinv(acc_sb)

    mkern = _make_gcn_msg_kernel()
    zeros32 = jnp.zeros((_CNA // 16, _C32), jnp.float32)
    h = x
    convs = [(W10, b10), (W11, b11)]
    convs_r = [(Wr0, br0), (Wr1, br1)]
    for i in range(2):
        ya = _ymat(h, convs[i][0], dinv_a)
        yb = _ymat(h, convs_r[i][0], dinv_b)
        (acc_a,) = mkern(row2, col2, ya, zeros32)
        (acc_b,) = mkern(row2r, col2r, yb, zeros32)
        ha = _comb(acc_a, ya, dinv_a, lw_a, convs[i][1])
        hb = _comb(acc_b, yb, dinv_b, lw_b, convs_r[i][1])
        h = ha + hb

    vm = jnp.where(pos_mask[:, None], x, 0.0)
    x1 = _proj(vm, A1, c1)
    x2 = _proj(vm, A2, c2)
    E = ei.shape[1]
    row = ei[0].astype(jnp.int32)
    col = ei[1].astype(jnp.int32)
    u = jnp.concatenate([x1[:E], jnp.ones((E, 1), jnp.float32)], axis=1)
    v = jnp.concatenate([x2[:E], jnp.ones((E, 1), jnp.float32)], axis=1)
    pi = pred_links[0].astype(jnp.int32)
    pk = pred_links[1].astype(jnp.int32)
    res, ind_f = _pair_products_sc(u, v, row, col, pi, pk, n)
    value = _value_head(res, ind_f, A3, c3)
    return _fold_head(h, value, A4, c4)


# speedup vs baseline: 47.2209x; 1.0107x over previous
import functools

import jax
import jax.numpy as jnp
from jax import lax
from jax.experimental import pallas as pl
from jax.experimental.pallas import tpu as pltpu
from jax.experimental.pallas import tpu_sc as plsc

N_NODE = 10000


def _ln(x, eps=1e-5):
    mu = jnp.mean(x, axis=-1, keepdims=True)
    var = jnp.mean((x - mu) ** 2, axis=-1, keepdims=True)
    return (x - mu) / jnp.sqrt(var + eps)


def _gcn_structure(edge, N):
    row = edge[0]
    col = edge[1]
    loop_idx = jnp.where(row == col, row, N)
    has_loop = jnp.zeros((N,), bool).at[loop_idx].set(True, mode='drop')
    loop_w = jnp.where(has_loop, 0.0, 1.0).astype(jnp.float32)
    deg = jnp.zeros((N,), jnp.float32).at[col].add(1.0) + loop_w
    dinv = jnp.where(deg > 0, 1.0 / jnp.sqrt(deg), 0.0)
    all_nodes = jnp.arange(N, dtype=row.dtype)
    row_f = jnp.concatenate([row, all_nodes])
    col_f = jnp.concatenate([col, all_nodes])
    norm_f = jnp.concatenate([dinv[row] * dinv[col], dinv * dinv * loop_w]).astype(jnp.float32)
    return row_f, col_f, norm_f


def _gcn(x, struct, W, b, N):
    row, col, norm = struct
    xw = x @ W
    msg = xw[row] * norm[:, None]
    return jax.ops.segment_sum(msg, col, num_segments=N) + b


def _pair_products_sparse(u, v, row, col, pi, pk, n):
    E = row.shape[0]
    code = row.astype(jnp.int32) * n + col.astype(jnp.int32)
    order = jnp.argsort(code)
    scode = code[order]
    scol = col.astype(jnp.int32)[order]
    us = u[order]
    vs = v[order]
    indptr = jnp.searchsorted(scode, jnp.arange(n + 1, dtype=jnp.int32) * n).astype(jnp.int32)
    deg = indptr[1:] - indptr[:-1]
    lo = indptr[pi]
    hi = indptr[pi + 1]
    P = pi.shape[0]

    def body(t, acc):
        e1 = jnp.minimum(lo + t, E - 1)
        active = (lo + t) < hi
        j = scol[e1]
        target = j * n + pk
        pos = jnp.clip(jnp.searchsorted(scode, target), 0, E - 1)
        found = active & (scode[pos] == target)
        prod = us[e1] * vs[pos] * found[:, None].astype(jnp.float32)
        return acc + prod

    maxdeg = jnp.max(deg)
    acc0 = jnp.zeros((P, u.shape[1]), jnp.float32)
    acc = jax.lax.fori_loop(0, maxdeg, body, acc0)
    return acc


_NW = 32
_CH = 256
_MB = 1024
_C32 = 32


def _make_pp_kernel(E, P, n):
    assert E % 2 == 0 and P % _CH == 0
    nchunk = P // _CH
    iptr_pad = ((n + 1 + 7) // 8) * 8
    i16 = lambda: lax.iota(jnp.int32, 16)

    def colval(colpk_v, e):
        word = plsc.load_gather(colpk_v, [e >> 1])
        return (word >> ((e & 1) << 4)) & 0xFFFF

    def lower_bound(colpk_v, l0, ln0, key):
        def body(_, c):
            lo_, ln_ = c
            half = ln_ >> 1
            mid = lo_ + half
            cm = colval(colpk_v, jnp.minimum(mid, E - 1))
            less = cm < key
            lo2 = jnp.where(less, mid + 1, lo_)
            ln2 = jnp.where(less, ln_ - half - 1, half)
            return (lo2, ln2)

        lo_f, _ = lax.fori_loop(0, 14, body, (l0, ln0))
        return lo_f

    def body(colpk_h, iptr_h, pi_h, pk_h, us_h, vs_h, res_h, ind_h,
             colpk_v, iptr_v, pi_v, pk_v, res_v, ind_v,
             m_pl, m_e1, m_e2, ubuf, vbuf, cnt_s, sem1, sem2):
        wid = lax.axis_index("s") * 2 + lax.axis_index("c")
        pltpu.sync_copy(colpk_h, colpk_v)
        pltpu.sync_copy(iptr_h, iptr_v)
        cnt_s[0] = 0
        zi = jnp.zeros((16,), jnp.int32)

        def init_mb(z, _):
            m_pl[pl.ds(z * 16, 16)] = zi
            m_e1[pl.ds(z * 16, 16)] = zi
            m_e2[pl.ds(z * 16, 16)] = zi
            return 0

        lax.fori_loop(0, _MB // 16, init_mb, 0)

        def flush():
            cnt = cnt_s[0]
            nb = (cnt + 127) // 128

            def fbody(b, _):
                off = b * 128
                d1 = pltpu.async_copy(us_h.at[m_e1.at[pl.ds(off, 128)]], ubuf, sem1)
                d2 = pltpu.async_copy(vs_h.at[m_e2.at[pl.ds(off, 128)]], vbuf, sem2)
                d1.wait()
                d2.wait()
                for g in range(8):
                    valid = (off + g * 16 + i16()) < cnt
                    plv = m_pl[pl.ds(off + g * 16, 16)]
                    midx = g * 16 + i16()

                    def cbody(c, _):
                        cs = jnp.zeros((16,), jnp.int32) + c
                        uv = plsc.load_gather(ubuf, [midx, cs])
                        vv = plsc.load_gather(vbuf, [midx, cs])
                        plsc.addupdate_scatter(res_v, [plv * _C32 + c], uv * vv,
                                               mask=valid)
                        return 0

                    lax.fori_loop(0, _C32, cbody, 0)
                return 0

            lax.fori_loop(0, nb, fbody, 0)
            cnt_s[0] = 0

        def chunk_body(it, _):
            cbase = (wid + it * _NW) * _CH
            pltpu.sync_copy(pi_h.at[pl.ds(cbase, _CH)], pi_v)
            pltpu.sync_copy(pk_h.at[pl.ds(cbase, _CH)], pk_v)

            def zbody(z, _):
                res_v[pl.ds(z * 16, 16)] = jnp.zeros((16,), jnp.float32)
                return 0

            lax.fori_loop(0, _CH * _C32 // 16, zbody, 0)

            def gloop(grp, _):
                lb = grp * 16
                vi = pi_v[pl.ds(lb, 16)]
                vk = pk_v[pl.ds(lb, 16)]
                lo = plsc.load_gather(iptr_v, [vi])
                hi = plsc.load_gather(iptr_v, [vi + 1])
                deg = hi - lo
                pos = lower_bound(colpk_v, lo, deg, vk)
                fnd = ((pos < hi)
                       & (colval(colpk_v, jnp.minimum(pos, E - 1)) == vk))
                ind_v[pl.ds(lb, 16)] = jnp.where(fnd, 1.0, 0.0)
                dmax = jnp.max(deg)

                def tbody(t, _):
                    @pl.when(cnt_s[0] > _MB - 16)
                    def _():
                        flush()

                    e1 = jnp.minimum(lo + t, E - 1)
                    act = t < deg
                    j = colval(colpk_v, e1)
                    jlo = plsc.load_gather(iptr_v, [j])
                    jhi = plsc.load_gather(iptr_v, [j + 1])
                    ln0 = jnp.where(act, jhi - jlo, 0)
                    pos2 = lower_bound(colpk_v, jlo, ln0, vk)
                    e2 = jnp.minimum(pos2, E - 1)
                    fd = act & (pos2 < jhi) & (colval(colpk_v, e2) == vk)
                    cnt = cnt_s[0]
                    plsc.store_compressed(m_pl.at[pl.ds(cnt, 16)], lb + i16(),
                                          mask=fd)
                    plsc.store_compressed(m_e1.at[pl.ds(cnt, 16)], e1, mask=fd)
                    plsc.store_compressed(m_e2.at[pl.ds(cnt, 16)], e2, mask=fd)
                    cnt_s[0] = cnt + jnp.sum(jnp.where(fd, 1, 0))
                    return 0

                lax.fori_loop(0, dmax, tbody, 0)
                return 0

            lax.fori_loop(0, _CH // 16, gloop, 0)
            flush()
            pltpu.sync_copy(res_v, res_h.at[pl.ds(cbase * _C32, _CH * _C32)])
            pltpu.sync_copy(ind_v, ind_h.at[pl.ds(cbase, _CH)])
            return 0

        niter = (nchunk - wid + _NW - 1) // _NW
        lax.fori_loop(0, niter, chunk_body, 0)

    mesh = plsc.VectorSubcoreMesh(core_axis_name="c", subcore_axis_name="s")
    return pl.kernel(
        body,
        out_type=[
            jax.ShapeDtypeStruct((P * _C32,), jnp.float32),
            jax.ShapeDtypeStruct((P,), jnp.float32),
        ],
        mesh=mesh,
        compiler_params=pltpu.CompilerParams(needs_layout_passes=False,
                                             use_tc_tiling_on_sc=False),
        scratch_types=[
            pltpu.VMEM((E // 2,), jnp.int32),
            pltpu.VMEM((iptr_pad,), jnp.int32),
            pltpu.VMEM((_CH,), jnp.int32),
            pltpu.VMEM((_CH,), jnp.int32),
            pltpu.VMEM((_CH * _C32,), jnp.float32),
            pltpu.VMEM((_CH,), jnp.float32),
            pltpu.VMEM((_MB,), jnp.int32),
            pltpu.VMEM((_MB,), jnp.int32),
            pltpu.VMEM((_MB,), jnp.int32),
            pltpu.VMEM((128, _C32), jnp.float32),
            pltpu.VMEM((128, _C32), jnp.float32),
            pltpu.SMEM((8,), jnp.int32),
            pltpu.SemaphoreType.DMA,
            pltpu.SemaphoreType.DMA,
        ],
    )


def _pair_products_sc(u, v, row, col, pi, pk, n):
    E = row.shape[0]
    P = pi.shape[0]
    code = row * n + col
    order = jnp.argsort(code)
    scode = code[order]
    scol = col[order]
    us = jnp.zeros((E, _C32), jnp.float32).at[:, : u.shape[1]].set(u[order])
    vs = jnp.zeros((E, _C32), jnp.float32).at[:, : v.shape[1]].set(v[order])
    colpk = scol[0::2] | (scol[1::2] << 16)
    iptr_pad = ((n + 1 + 7) // 8) * 8
    iptr = jnp.full((iptr_pad,), E, jnp.int32).at[: n + 1].set(
        jnp.searchsorted(scode, jnp.arange(n + 1, dtype=jnp.int32) * n)
        .astype(jnp.int32))
    res_flat, ind = _make_pp_kernel(E, P, n)(colpk, iptr, pi, pk, us, vs)
    return res_flat.reshape(P, _C32), ind


_M2 = 2560000
_ET = _M2 // _NW
_B1 = 2000
_NGB = _B1 // 16
_NBK = _ET // _B1
_CN = 40704
_CNA = 40960
_NCHK = 4
_PEND = 1280
_FLUSH = 1152
_NFB = _FLUSH // 128
_NS_ROWS = 160000
_NSA = 160256


def _make_gcn_msg_kernel():
    rows_pt = _CNA // 16
    wb_pt = _CN // 16
    dump = _CN

    def body(row_h, col_h, y_h, zeros_h, acc_h,
             row_v, col_v, pend_r, pend_c, ybuf, acc_sp, ps, sem1, sem2):
        cid = lax.axis_index("c")
        sid = lax.axis_index("s")
        wid = sid * 2 + cid
        i16 = lax.iota(jnp.int32, 16)
        eb = wid * _ET

        def init_pend(z, _):
            pend_r[pl.ds(z * 16, 16)] = jnp.zeros((16,), jnp.int32)
            pend_c[pl.ds(z * 16, 16)] = jnp.zeros((16,), jnp.int32) + dump
            return 0

        lax.fori_loop(0, _PEND // 16, init_pend, 0)

        for ch in range(_NCHK):
            base = ch * _CN
            pltpu.sync_copy(zeros_h, acc_sp.at[pl.ds(sid * rows_pt, rows_pt), :])
            plsc.subcore_barrier()
            ps[0] = 0

            def flush3():
                ds_g = []
                for b in range(_NFB):
                    ds_g.append(pltpu.async_copy(
                        y_h.at[pend_r.at[pl.ds(b * 128, 128)]],
                        ybuf.at[pl.ds(b * 128, 128), :], sem1))
                for d in ds_g:
                    d.wait()
                ds_s = []
                for b in range(_NFB):
                    ds_s.append(pltpu.async_copy(
                        ybuf.at[pl.ds(b * 128, 128), :],
                        acc_sp.at[pend_c.at[pl.ds(b * 128, 128)]], sem2,
                        add=True))
                for d in ds_s:
                    d.wait()
                pend_r[pl.ds(0, 16)] = pend_r[pl.ds(_FLUSH, 16)]
                pend_c[pl.ds(0, 16)] = pend_c[pl.ds(_FLUSH, 16)]
                ps[0] = ps[0] - _FLUSH

            def blk(bk, _):
                off = eb + bk * _B1
                pltpu.sync_copy(row_h.at[pl.ds(off, _B1)], row_v)
                pltpu.sync_copy(col_h.at[pl.ds(off, _B1)], col_v)

                def grp(g, _):
                    r16 = row_v[pl.ds(g * 16, 16)]
                    c16 = col_v[pl.ds(g * 16, 16)]
                    m = (c16 >= base) & (c16 < base + _CN)
                    pcnt = ps[0]
                    plsc.store_compressed(pend_r.at[pl.ds(pcnt, 16)], r16,
                                          mask=m)
                    plsc.store_compressed(pend_c.at[pl.ds(pcnt, 16)],
                                          c16 - base, mask=m)
                    ps[0] = pcnt + jnp.sum(jnp.where(m, 1, 0))

                    @pl.when(ps[0] >= _FLUSH)
                    def _():
                        flush3()

                    return 0

                lax.fori_loop(0, _NGB, grp, 0)
                return 0

            lax.fori_loop(0, _NBK, blk, 0)

            pcnt = ps[0]

            def padg(g, _):
                idx = g * 16 + i16
                plsc.store_scatter(pend_c, [idx],
                                   jnp.zeros((16,), jnp.int32) + dump,
                                   mask=idx >= pcnt)
                return 0

            lax.fori_loop(0, _FLUSH // 16, padg, 0)
            nb = (pcnt + 127) // 128

            def dr(b, _):
                pltpu.async_copy(
                    y_h.at[pend_r.at[pl.ds(b * 128, 128)]],
                    ybuf.at[pl.ds(b * 128, 128), :], sem1).wait()
                pltpu.sync_copy(ybuf.at[pl.ds(b * 128, 128), :],
                                acc_sp.at[pend_c.at[pl.ds(b * 128, 128)]],
                                add=True)
                return 0

            lax.fori_loop(0, nb, dr, 0)
            ps[0] = 0
            plsc.subcore_barrier()
            pltpu.sync_copy(
                acc_sp.at[pl.ds(sid * wb_pt, wb_pt), :],
                acc_h.at[cid, pl.ds(base + sid * wb_pt, wb_pt), :])
            plsc.subcore_barrier()

    mesh = plsc.VectorSubcoreMesh(core_axis_name="c", subcore_axis_name="s")
    return pl.kernel(
        body,
        out_type=[jax.ShapeDtypeStruct((2, _NCHK * _CN, _C32), jnp.float32)],
        mesh=mesh,
        compiler_params=pltpu.CompilerParams(needs_layout_passes=False,
                                             use_tc_tiling_on_sc=False),
        scratch_types=[
            pltpu.VMEM((_B1,), jnp.int32),
            pltpu.VMEM((_B1,), jnp.int32),
            pltpu.VMEM((_PEND,), jnp.int32),
            pltpu.VMEM((_PEND,), jnp.int32),
            pltpu.VMEM((_FLUSH, _C32), jnp.float32),
            pltpu.VMEM_SHARED((_CNA, _C32), jnp.float32),
            pltpu.SMEM((8,), jnp.int32),
            pltpu.SemaphoreType.DMA,
            pltpu.SemaphoreType.DMA,
        ],
    )


def _make_gcn_struct_kernel():
    rows_pt = _NSA // 16
    wb_pt = _NS_ROWS // 16
    dump = _NS_ROWS

    def body(row_h, col_h, zeros_h, acc_h,
             row_v, col_v, pend_c, pend_il, pay, acc_sp, ps, sem2):
        cid = lax.axis_index("c")
        sid = lax.axis_index("s")
        wid = sid * 2 + cid
        i16 = lax.iota(jnp.int32, 16)
        eb = wid * _ET
        tru = jnp.ones((16,), jnp.bool_)

        def init_pend(z, _):
            pend_c[pl.ds(z * 16, 16)] = jnp.zeros((16,), jnp.int32) + dump
            pend_il[pl.ds(z * 16, 16)] = jnp.zeros((16,), jnp.float32)
            return 0

        lax.fori_loop(0, _PEND // 16, init_pend, 0)
        zrow = jnp.zeros((16,), jnp.float32)
        pltpu.sync_copy(zeros_h.at[pl.ds(0, _FLUSH), :], pay)

        def set_ones(g, _):
            plsc.store_scatter(pay, [g * 16 + i16, i16 * 0], zrow + 1.0,
                               mask=tru)
            return 0

        lax.fori_loop(0, _FLUSH // 16, set_ones, 0)

        pltpu.sync_copy(zeros_h, acc_sp.at[pl.ds(sid * rows_pt, rows_pt), :])
        plsc.subcore_barrier()
        ps[0] = 0

        def expand_il(nslots):
            def ex(g, _):
                il = pend_il[pl.ds(g * 16, 16)]
                plsc.store_scatter(pay, [g * 16 + i16, i16 * 0 + 1], il,
                                   mask=tru)
                return 0

            lax.fori_loop(0, nslots // 16, ex, 0)

        def flush3():
            expand_il(_FLUSH)
            ds_s = []
            for b in range(_NFB):
                ds_s.append(pltpu.async_copy(
                    pay.at[pl.ds(b * 128, 128), :],
                    acc_sp.at[pend_c.at[pl.ds(b * 128, 128)]], sem2,
                    add=True))
            for d in ds_s:
                d.wait()
            pend_c[pl.ds(0, 16)] = pend_c[pl.ds(_FLUSH, 16)]
            pend_il[pl.ds(0, 16)] = pend_il[pl.ds(_FLUSH, 16)]
            ps[0] = ps[0] - _FLUSH

        def blk(bk, _):
            off = eb + bk * _B1
            pltpu.sync_copy(row_h.at[pl.ds(off, _B1)], row_v)
            pltpu.sync_copy(col_h.at[pl.ds(off, _B1)], col_v)

            def grp(g, _):
                r16 = row_v[pl.ds(g * 16, 16)]
                c16 = col_v[pl.ds(g * 16, 16)]
                il = jnp.where(r16 == c16, 1.0, 0.0)
                pcnt = ps[0]
                plsc.store_compressed(pend_c.at[pl.ds(pcnt, 16)], c16,
                                      mask=tru)
                plsc.store_compressed(pend_il.at[pl.ds(pcnt, 16)], il,
                                      mask=tru)
                ps[0] = pcnt + 16

                @pl.when(ps[0] >= _FLUSH)
                def _():
                    flush3()

                return 0

            lax.fori_loop(0, _NGB, grp, 0)
            return 0

        lax.fori_loop(0, _NBK, blk, 0)

        pcnt = ps[0]

        def padg(g, _):
            idx = g * 16 + i16
            plsc.store_scatter(pend_c, [idx],
                               jnp.zeros((16,), jnp.int32) + dump,
                               mask=idx >= pcnt)
            plsc.store_scatter(pend_il, [idx], jnp.zeros((16,), jnp.float32),
                               mask=idx >= pcnt)
            return 0

        lax.fori_loop(0, _FLUSH // 16, padg, 0)
        expand_il(_FLUSH)
        nb = (pcnt + 127) // 128

        def dr(b, _):
            pltpu.sync_copy(pay.at[pl.ds(b * 128, 128), :],
                            acc_sp.at[pend_c.at[pl.ds(b * 128, 128)]],
                            add=True)
            return 0

        lax.fori_loop(0, nb, dr, 0)
        plsc.subcore_barrier()
        pltpu.sync_copy(
            acc_sp.at[pl.ds(sid * wb_pt, wb_pt), :],
            acc_h.at[cid, pl.ds(sid * wb_pt, wb_pt), :])

    mesh = plsc.VectorSubcoreMesh(core_axis_name="c", subcore_axis_name="s")
    return pl.kernel(
        body,
        out_type=[jax.ShapeDtypeStruct((2, _NS_ROWS, 8), jnp.float32)],
        mesh=mesh,
        compiler_params=pltpu.CompilerParams(needs_layout_passes=False,
                                             use_tc_tiling_on_sc=False),
        scratch_types=[
            pltpu.VMEM((_B1,), jnp.int32),
            pltpu.VMEM((_B1,), jnp.int32),
            pltpu.VMEM((_PEND,), jnp.int32),
            pltpu.VMEM((_PEND,), jnp.float32),
            pltpu.VMEM((_FLUSH, 8), jnp.float32),
            pltpu.VMEM_SHARED((_NSA, 8), jnp.float32),
            pltpu.SMEM((8,), jnp.int32),
            pltpu.SemaphoreType.DMA,
        ],
    )


_BR = 2000


def _ln_rows(x, eps=1e-5):
    mu = jnp.mean(x, axis=-1, keepdims=True)
    var = jnp.mean((x - mu) ** 2, axis=-1, keepdims=True)
    return (x - mu) / jnp.sqrt(var + eps)


def _ymat_kernel(h_ref, w_ref, dinv_ref, o_ref):
    o_ref[...] = dinv_ref[...] * jnp.dot(
        h_ref[...], w_ref[...], preferred_element_type=jnp.float32)


def _ymat(h, W, dinv):
    N, K = h.shape
    F = W.shape[1]
    return pl.pallas_call(
        _ymat_kernel,
        grid=(N // _BR,),
        in_specs=[
            pl.BlockSpec((_BR, K), lambda i: (i, 0)),
            pl.BlockSpec((K, F), lambda i: (0, 0)),
            pl.BlockSpec((_BR, 1), lambda i: (i, 0)),
        ],
        out_specs=pl.BlockSpec((_BR, F), lambda i: (i, 0)),
        out_shape=jax.ShapeDtypeStruct((N, F), jnp.float32),
    )(h, W, dinv)


def _comb_kernel(acc_ref, y_ref, dinv_ref, lw_ref, b_ref, o_ref):
    a = acc_ref[0] + acc_ref[1]
    o = dinv_ref[...] * (a + lw_ref[...] * y_ref[...]) + b_ref[...]
    o_ref[...] = jax.nn.relu(_ln_rows(o))


def _comb(acc, y, dinv, lw, b):
    N, F = y.shape
    return pl.pallas_call(
        _comb_kernel,
        grid=(N // _BR,),
        in_specs=[
            pl.BlockSpec((2, _BR, F), lambda i: (0, i, 0)),
            pl.BlockSpec((_BR, F), lambda i: (i, 0)),
            pl.BlockSpec((_BR, 1), lambda i: (i, 0)),
            pl.BlockSpec((_BR, 1), lambda i: (i, 0)),
            pl.BlockSpec((1, F), lambda i: (0, 0)),
        ],
        out_specs=pl.BlockSpec((_BR, F), lambda i: (i, 0)),
        out_shape=jax.ShapeDtypeStruct((N, F), jnp.float32),
    )(acc, y, dinv, lw, b.reshape(1, -1))


def _dinv_kernel(acc_ref, dinv_ref, lw_ref):
    cnt = acc_ref[0, :, 0] + acc_ref[1, :, 0]
    il = acc_ref[0, :, 1] + acc_ref[1, :, 1]
    lw = jnp.where(il > 0, 0.0, 1.0)
    deg = cnt + lw
    dinv_ref[...] = jnp.where(deg > 0, jax.lax.rsqrt(jnp.maximum(deg, 1e-30)),
                              0.0)[:, None]
    lw_ref[...] = lw[:, None]


def _dinv(acc):
    N = acc.shape[1]
    return pl.pallas_call(
        _dinv_kernel,
        grid=(N // _BR,),
        in_specs=[pl.BlockSpec((2, _BR, 8), lambda i: (0, i, 0))],
        out_specs=[
            pl.BlockSpec((_BR, 1), lambda i: (i, 0)),
            pl.BlockSpec((_BR, 1), lambda i: (i, 0)),
        ],
        out_shape=[
            jax.ShapeDtypeStruct((N, 1), jnp.float32),
            jax.ShapeDtypeStruct((N, 1), jnp.float32),
        ],
    )(acc)


def _proj_kernel(x_ref, a_ref, c_ref, o_ref):
    o_ref[...] = jax.nn.relu(_ln_rows(
        jnp.dot(x_ref[...], a_ref[...], preferred_element_type=jnp.float32)
        + c_ref[...]))


def _proj(x, A, c):
    N, K = x.shape
    F = A.shape[1]
    return pl.pallas_call(
        _proj_kernel,
        grid=(N // _BR,),
        in_specs=[
            pl.BlockSpec((_BR, K), lambda i: (i, 0)),
            pl.BlockSpec((K, F), lambda i: (0, 0)),
            pl.BlockSpec((1, F), lambda i: (0, 0)),
        ],
        out_specs=pl.BlockSpec((_BR, F), lambda i: (i, 0)),
        out_shape=jax.ShapeDtypeStruct((N, F), jnp.float32),
    )(x, A, c.reshape(1, -1))


def _value_kernel(res_ref, ind_ref, a3p_ref, a3i_ref, c3_ref, o_ref):
    res = res_ref[...]
    ind = ind_ref[...]
    o = (jnp.dot(res, a3p_ref[...], preferred_element_type=jnp.float32)
         + ind * a3i_ref[...] + c3_ref[...])
    o = jax.nn.relu(_ln_rows(o))
    keep = (res[:, 24:25] > 0) | (ind > 0.5)
    o_ref[...] = jnp.where(keep, o, 0.0)


def _value_head(res, ind, A3, c3):
    P = res.shape[0]
    F = A3.shape[1]
    a3p = jnp.zeros((_C32, F), jnp.float32).at[:24].set(A3[:24])
    a3i = A3[24].reshape(1, F)
    return pl.pallas_call(
        _value_kernel,
        grid=(P // _BR,),
        in_specs=[
            pl.BlockSpec((_BR, _C32), lambda i: (i, 0)),
            pl.BlockSpec((_BR, 1), lambda i: (i, 0)),
            pl.BlockSpec((_C32, F), lambda i: (0, 0)),
            pl.BlockSpec((1, F), lambda i: (0, 0)),
            pl.BlockSpec((1, F), lambda i: (0, 0)),
        ],
        out_specs=pl.BlockSpec((_BR, F), lambda i: (i, 0)),
        out_shape=jax.ShapeDtypeStruct((P, F), jnp.float32),
    )(res, ind.reshape(P, 1), a3p, a3i, c3.reshape(1, -1))


def _fold_head_kernel(ht_ref, hb_ref, vt_ref, vb_ref, a4_ref, c4_ref, o_ref):
    h = jnp.concatenate([ht_ref[...] + hb_ref[...],
                         vt_ref[...] + vb_ref[...]], axis=1)
    o_ref[...] = jax.nn.sigmoid(
        jnp.dot(h, a4_ref[...], preferred_element_type=jnp.float32)
        + c4_ref[...])


def _fold_head(hg, value, A4, c4):
    N, F1 = hg.shape
    F2 = value.shape[1]
    R = N // 2
    G = R // _BR
    return pl.pallas_call(
        _fold_head_kernel,
        grid=(G,),
        in_specs=[
            pl.BlockSpec((_BR, F1), lambda i: (i, 0)),
            pl.BlockSpec((_BR, F1), lambda i: (i + G, 0)),
            pl.BlockSpec((_BR, F2), lambda i: (i, 0)),
            pl.BlockSpec((_BR, F2), lambda i: (i + G, 0)),
            pl.BlockSpec((F1 + F2, A4.shape[1]), lambda i: (0, 0)),
            pl.BlockSpec((1, A4.shape[1]), lambda i: (0, 0)),
        ],
        out_specs=pl.BlockSpec((_BR, A4.shape[1]), lambda i: (i, 0)),
        out_shape=jax.ShapeDtypeStruct((R, A4.shape[1]), jnp.float32),
    )(hg, hg, value, value, A4, c4.reshape(1, -1))


def kernel(x, W10, b10, W11, b11, Wr0, br0, Wr1, br1, A1, c1, A2, c2, A3, c3, A4, c4, edge2, edge2_r, ei, pred_links, num_node, pos_mask):
    n = N_NODE
    row2 = edge2[0].astype(jnp.int32)
    col2 = edge2[1].astype(jnp.int32)
    row2r = edge2_r[0].astype(jnp.int32)
    col2r = edge2_r[1].astype(jnp.int32)

    skern = _make_gcn_struct_kernel()
    zeros8 = jnp.zeros((_NSA // 16, 8), jnp.float32)
    (acc_sa,) = skern(row2, col2, zeros8)
    (acc_sb,) = skern(row2r, col2r, zeros8)
    dinv_a, lw_a = _dinv(acc_sa)
    dinv_b, lw_b = _dinv(acc_sb)

    mkern = _make_gcn_msg_kernel()
    zeros32 = jnp.zeros((_CNA // 16, _C32), jnp.float32)
    h = x
    convs = [(W10, b10), (W11, b11)]
    convs_r = [(Wr0, br0), (Wr1, br1)]
    for i in range(2):
        ya = _ymat(h, convs[i][0], dinv_a)
        yb = _ymat(h, convs_r[i][0], dinv_b)
        (acc_a,) = mkern(row2, col2, ya, zeros32)
        (acc_b,) = mkern(row2r, col2r, yb, zeros32)
        ha = _comb(acc_a, ya, dinv_a, lw_a, convs[i][1])
        hb = _comb(acc_b, yb, dinv_b, lw_b, convs_r[i][1])
        h = ha + hb

    vm = jnp.where(pos_mask[:, None], x, 0.0)
    x1 = _proj(vm, A1, c1)
    x2 = _proj(vm, A2, c2)
    E = ei.shape[1]
    row = ei[0].astype(jnp.int32)
    col = ei[1].astype(jnp.int32)
    u = jnp.concatenate([x1[:E], jnp.ones((E, 1), jnp.float32)], axis=1)
    v = jnp.concatenate([x2[:E], jnp.ones((E, 1), jnp.float32)], axis=1)
    pi = pred_links[0].astype(jnp.int32)
    pk = pred_links[1].astype(jnp.int32)
    res, ind_f = _pair_products_sc(u, v, row, col, pi, pk, n)
    value = _value_head(res, ind_f, A3, c3)
    return _fold_head(h, value, A4, c4)


# speedup vs baseline: 75.1944x; 1.5924x over previous
import functools

import jax
import jax.numpy as jnp
from jax import lax
from jax.experimental import pallas as pl
from jax.experimental.pallas import tpu as pltpu
from jax.experimental.pallas import tpu_sc as plsc

N_NODE = 10000


def _ln(x, eps=1e-5):
    mu = jnp.mean(x, axis=-1, keepdims=True)
    var = jnp.mean((x - mu) ** 2, axis=-1, keepdims=True)
    return (x - mu) / jnp.sqrt(var + eps)


def _gcn_structure(edge, N):
    row = edge[0]
    col = edge[1]
    loop_idx = jnp.where(row == col, row, N)
    has_loop = jnp.zeros((N,), bool).at[loop_idx].set(True, mode='drop')
    loop_w = jnp.where(has_loop, 0.0, 1.0).astype(jnp.float32)
    deg = jnp.zeros((N,), jnp.float32).at[col].add(1.0) + loop_w
    dinv = jnp.where(deg > 0, 1.0 / jnp.sqrt(deg), 0.0)
    all_nodes = jnp.arange(N, dtype=row.dtype)
    row_f = jnp.concatenate([row, all_nodes])
    col_f = jnp.concatenate([col, all_nodes])
    norm_f = jnp.concatenate([dinv[row] * dinv[col], dinv * dinv * loop_w]).astype(jnp.float32)
    return row_f, col_f, norm_f


def _gcn(x, struct, W, b, N):
    row, col, norm = struct
    xw = x @ W
    msg = xw[row] * norm[:, None]
    return jax.ops.segment_sum(msg, col, num_segments=N) + b


def _pair_products_sparse(u, v, row, col, pi, pk, n):
    E = row.shape[0]
    code = row.astype(jnp.int32) * n + col.astype(jnp.int32)
    order = jnp.argsort(code)
    scode = code[order]
    scol = col.astype(jnp.int32)[order]
    us = u[order]
    vs = v[order]
    indptr = jnp.searchsorted(scode, jnp.arange(n + 1, dtype=jnp.int32) * n).astype(jnp.int32)
    deg = indptr[1:] - indptr[:-1]
    lo = indptr[pi]
    hi = indptr[pi + 1]
    P = pi.shape[0]

    def body(t, acc):
        e1 = jnp.minimum(lo + t, E - 1)
        active = (lo + t) < hi
        j = scol[e1]
        target = j * n + pk
        pos = jnp.clip(jnp.searchsorted(scode, target), 0, E - 1)
        found = active & (scode[pos] == target)
        prod = us[e1] * vs[pos] * found[:, None].astype(jnp.float32)
        return acc + prod

    maxdeg = jnp.max(deg)
    acc0 = jnp.zeros((P, u.shape[1]), jnp.float32)
    acc = jax.lax.fori_loop(0, maxdeg, body, acc0)
    return acc


_NW = 32
_CH = 256
_MB = 1024
_C32 = 32


def _make_pp_kernel(E, P, n):
    assert E % 2 == 0 and P % _CH == 0
    nchunk = P // _CH
    iptr_pad = ((n + 1 + 7) // 8) * 8
    i16 = lambda: lax.iota(jnp.int32, 16)

    def colval(colpk_v, e):
        word = plsc.load_gather(colpk_v, [e >> 1])
        return (word >> ((e & 1) << 4)) & 0xFFFF

    def lower_bound(colpk_v, l0, ln0, key):
        def body(_, c):
            lo_, ln_ = c
            half = ln_ >> 1
            mid = lo_ + half
            cm = colval(colpk_v, jnp.minimum(mid, E - 1))
            less = cm < key
            lo2 = jnp.where(less, mid + 1, lo_)
            ln2 = jnp.where(less, ln_ - half - 1, half)
            return (lo2, ln2)

        lo_f, _ = lax.fori_loop(0, 14, body, (l0, ln0))
        return lo_f

    def body(colpk_h, iptr_h, pi_h, pk_h, us_h, vs_h, res_h, ind_h,
             colpk_v, iptr_v, pi_v, pk_v, res_v, ind_v,
             m_pl, m_e1, m_e2, ubuf, vbuf, cnt_s, sem1, sem2):
        wid = lax.axis_index("s") * 2 + lax.axis_index("c")
        pltpu.sync_copy(colpk_h, colpk_v)
        pltpu.sync_copy(iptr_h, iptr_v)
        cnt_s[0] = 0
        zi = jnp.zeros((16,), jnp.int32)

        def init_mb(z, _):
            m_pl[pl.ds(z * 16, 16)] = zi
            m_e1[pl.ds(z * 16, 16)] = zi
            m_e2[pl.ds(z * 16, 16)] = zi
            return 0

        lax.fori_loop(0, _MB // 16, init_mb, 0)

        def flush():
            cnt = cnt_s[0]
            nb = (cnt + 127) // 128

            def fbody(b, _):
                off = b * 128
                d1 = pltpu.async_copy(us_h.at[m_e1.at[pl.ds(off, 128)]], ubuf, sem1)
                d2 = pltpu.async_copy(vs_h.at[m_e2.at[pl.ds(off, 128)]], vbuf, sem2)
                d1.wait()
                d2.wait()
                for g in range(8):
                    valid = (off + g * 16 + i16()) < cnt
                    plv = m_pl[pl.ds(off + g * 16, 16)]
                    midx = g * 16 + i16()

                    def cbody(c, _):
                        cs = jnp.zeros((16,), jnp.int32) + c
                        uv = plsc.load_gather(ubuf, [midx, cs])
                        vv = plsc.load_gather(vbuf, [midx, cs])
                        plsc.addupdate_scatter(res_v, [plv * _C32 + c], uv * vv,
                                               mask=valid)
                        return 0

                    lax.fori_loop(0, _C32, cbody, 0)
                return 0

            lax.fori_loop(0, nb, fbody, 0)
            cnt_s[0] = 0

        def chunk_body(it, _):
            cbase = (wid + it * _NW) * _CH
            pltpu.sync_copy(pi_h.at[pl.ds(cbase, _CH)], pi_v)
            pltpu.sync_copy(pk_h.at[pl.ds(cbase, _CH)], pk_v)

            def zbody(z, _):
                res_v[pl.ds(z * 16, 16)] = jnp.zeros((16,), jnp.float32)
                return 0

            lax.fori_loop(0, _CH * _C32 // 16, zbody, 0)

            def gloop(grp, _):
                lb = grp * 16
                vi = pi_v[pl.ds(lb, 16)]
                vk = pk_v[pl.ds(lb, 16)]
                lo = plsc.load_gather(iptr_v, [vi])
                hi = plsc.load_gather(iptr_v, [vi + 1])
                deg = hi - lo
                pos = lower_bound(colpk_v, lo, deg, vk)
                fnd = ((pos < hi)
                       & (colval(colpk_v, jnp.minimum(pos, E - 1)) == vk))
                ind_v[pl.ds(lb, 16)] = jnp.where(fnd, 1.0, 0.0)
                dmax = jnp.max(deg)

                def tbody(t, _):
                    @pl.when(cnt_s[0] > _MB - 16)
                    def _():
                        flush()

                    e1 = jnp.minimum(lo + t, E - 1)
                    act = t < deg
                    j = colval(colpk_v, e1)
                    jlo = plsc.load_gather(iptr_v, [j])
                    jhi = plsc.load_gather(iptr_v, [j + 1])
                    ln0 = jnp.where(act, jhi - jlo, 0)
                    pos2 = lower_bound(colpk_v, jlo, ln0, vk)
                    e2 = jnp.minimum(pos2, E - 1)
                    fd = act & (pos2 < jhi) & (colval(colpk_v, e2) == vk)
                    cnt = cnt_s[0]
                    plsc.store_compressed(m_pl.at[pl.ds(cnt, 16)], lb + i16(),
                                          mask=fd)
                    plsc.store_compressed(m_e1.at[pl.ds(cnt, 16)], e1, mask=fd)
                    plsc.store_compressed(m_e2.at[pl.ds(cnt, 16)], e2, mask=fd)
                    cnt_s[0] = cnt + jnp.sum(jnp.where(fd, 1, 0))
                    return 0

                lax.fori_loop(0, dmax, tbody, 0)
                return 0

            lax.fori_loop(0, _CH // 16, gloop, 0)
            flush()
            pltpu.sync_copy(res_v, res_h.at[pl.ds(cbase * _C32, _CH * _C32)])
            pltpu.sync_copy(ind_v, ind_h.at[pl.ds(cbase, _CH)])
            return 0

        niter = (nchunk - wid + _NW - 1) // _NW
        lax.fori_loop(0, niter, chunk_body, 0)

    mesh = plsc.VectorSubcoreMesh(core_axis_name="c", subcore_axis_name="s")
    return pl.kernel(
        body,
        out_type=[
            jax.ShapeDtypeStruct((P * _C32,), jnp.float32),
            jax.ShapeDtypeStruct((P,), jnp.float32),
        ],
        mesh=mesh,
        compiler_params=pltpu.CompilerParams(needs_layout_passes=False,
                                             use_tc_tiling_on_sc=False),
        scratch_types=[
            pltpu.VMEM((E // 2,), jnp.int32),
            pltpu.VMEM((iptr_pad,), jnp.int32),
            pltpu.VMEM((_CH,), jnp.int32),
            pltpu.VMEM((_CH,), jnp.int32),
            pltpu.VMEM((_CH * _C32,), jnp.float32),
            pltpu.VMEM((_CH,), jnp.float32),
            pltpu.VMEM((_MB,), jnp.int32),
            pltpu.VMEM((_MB,), jnp.int32),
            pltpu.VMEM((_MB,), jnp.int32),
            pltpu.VMEM((128, _C32), jnp.float32),
            pltpu.VMEM((128, _C32), jnp.float32),
            pltpu.SMEM((8,), jnp.int32),
            pltpu.SemaphoreType.DMA,
            pltpu.SemaphoreType.DMA,
        ],
    )


def _pair_products_sc(u, v, row, col, pi, pk, n):
    E = row.shape[0]
    P = pi.shape[0]
    code = row * n + col
    scode, order = lax.sort((code, lax.iota(jnp.int32, E)), num_keys=1,
                            is_stable=False)
    scol = scode % n
    us = jnp.zeros((E, _C32), jnp.float32).at[:, : u.shape[1]].set(u[order])
    vs = jnp.zeros((E, _C32), jnp.float32).at[:, : v.shape[1]].set(v[order])
    colpk = scol[0::2] | (scol[1::2] << 16)
    iptr_pad = ((n + 1 + 7) // 8) * 8
    iptr = jnp.full((iptr_pad,), E, jnp.int32).at[: n + 1].set(
        jnp.searchsorted(scode, jnp.arange(n + 1, dtype=jnp.int32) * n)
        .astype(jnp.int32))
    res_flat, ind = _make_pp_kernel(E, P, n)(colpk, iptr, pi, pk, us, vs)
    return res_flat.reshape(P, _C32), ind


_M2 = 2560000
_ET = _M2 // _NW
_B1 = 2000
_NGB = _B1 // 16
_NBK = _ET // _B1
_CN = 40704
_CNA = 40960
_NCHK = 4
_PEND = 1280
_FLUSH = 1152
_NFB = _FLUSH // 128
_NS_ROWS = 160000
_NSA = 160256


def _make_gcn_msg_kernel():
    rows_pt = _CNA // 16
    wb_pt = _CN // 16
    dump = _CN

    def body(row_h, col_h, y_h, zeros_h, acc_h,
             row_v, col_v, pend_r, pend_c, ybuf, acc_sp, ps, sem1, sem2):
        cid = lax.axis_index("c")
        sid = lax.axis_index("s")
        wid = sid * 2 + cid
        i16 = lax.iota(jnp.int32, 16)
        eb = wid * _ET

        def init_pend(z, _):
            pend_r[pl.ds(z * 16, 16)] = jnp.zeros((16,), jnp.int32)
            pend_c[pl.ds(z * 16, 16)] = jnp.zeros((16,), jnp.int32) + dump
            return 0

        lax.fori_loop(0, _PEND // 16, init_pend, 0)

        for ch in range(_NCHK):
            base = ch * _CN
            pltpu.sync_copy(zeros_h, acc_sp.at[pl.ds(sid * rows_pt, rows_pt), :])
            plsc.subcore_barrier()
            ps[0] = 0

            def flush3():
                ds_g = []
                for b in range(_NFB):
                    ds_g.append(pltpu.async_copy(
                        y_h.at[pend_r.at[pl.ds(b * 128, 128)]],
                        ybuf.at[pl.ds(b * 128, 128), :], sem1))
                for d in ds_g:
                    d.wait()
                ds_s = []
                for b in range(_NFB):
                    ds_s.append(pltpu.async_copy(
                        ybuf.at[pl.ds(b * 128, 128), :],
                        acc_sp.at[pend_c.at[pl.ds(b * 128, 128)]], sem2,
                        add=True))
                for d in ds_s:
                    d.wait()
                pend_r[pl.ds(0, 16)] = pend_r[pl.ds(_FLUSH, 16)]
                pend_c[pl.ds(0, 16)] = pend_c[pl.ds(_FLUSH, 16)]
                ps[0] = ps[0] - _FLUSH

            def blk(bk, _):
                off = eb + bk * _B1
                pltpu.sync_copy(row_h.at[pl.ds(off, _B1)], row_v)
                pltpu.sync_copy(col_h.at[pl.ds(off, _B1)], col_v)

                def grp(g, _):
                    r16 = row_v[pl.ds(g * 16, 16)]
                    c16 = col_v[pl.ds(g * 16, 16)]
                    m = (c16 >= base) & (c16 < base + _CN)
                    pcnt = ps[0]
                    plsc.store_compressed(pend_r.at[pl.ds(pcnt, 16)], r16,
                                          mask=m)
                    plsc.store_compressed(pend_c.at[pl.ds(pcnt, 16)],
                                          c16 - base, mask=m)
                    ps[0] = pcnt + jnp.sum(jnp.where(m, 1, 0))

                    @pl.when(ps[0] >= _FLUSH)
                    def _():
                        flush3()

                    return 0

                lax.fori_loop(0, _NGB, grp, 0)
                return 0

            lax.fori_loop(0, _NBK, blk, 0)

            pcnt = ps[0]

            def padg(g, _):
                idx = g * 16 + i16
                plsc.store_scatter(pend_c, [idx],
                                   jnp.zeros((16,), jnp.int32) + dump,
                                   mask=idx >= pcnt)
                return 0

            lax.fori_loop(0, _FLUSH // 16, padg, 0)
            nb = (pcnt + 127) // 128

            def dr(b, _):
                pltpu.async_copy(
                    y_h.at[pend_r.at[pl.ds(b * 128, 128)]],
                    ybuf.at[pl.ds(b * 128, 128), :], sem1).wait()
                pltpu.sync_copy(ybuf.at[pl.ds(b * 128, 128), :],
                                acc_sp.at[pend_c.at[pl.ds(b * 128, 128)]],
                                add=True)
                return 0

            lax.fori_loop(0, nb, dr, 0)
            ps[0] = 0
            plsc.subcore_barrier()
            pltpu.sync_copy(
                acc_sp.at[pl.ds(sid * wb_pt, wb_pt), :],
                acc_h.at[cid, pl.ds(base + sid * wb_pt, wb_pt), :])
            plsc.subcore_barrier()

    mesh = plsc.VectorSubcoreMesh(core_axis_name="c", subcore_axis_name="s")
    return pl.kernel(
        body,
        out_type=[jax.ShapeDtypeStruct((2, _NCHK * _CN, _C32), jnp.float32)],
        mesh=mesh,
        compiler_params=pltpu.CompilerParams(needs_layout_passes=False,
                                             use_tc_tiling_on_sc=False),
        scratch_types=[
            pltpu.VMEM((_B1,), jnp.int32),
            pltpu.VMEM((_B1,), jnp.int32),
            pltpu.VMEM((_PEND,), jnp.int32),
            pltpu.VMEM((_PEND,), jnp.int32),
            pltpu.VMEM((_FLUSH, _C32), jnp.float32),
            pltpu.VMEM_SHARED((_CNA, _C32), jnp.float32),
            pltpu.SMEM((8,), jnp.int32),
            pltpu.SemaphoreType.DMA,
            pltpu.SemaphoreType.DMA,
        ],
    )


def _make_gcn_struct_kernel():
    rows_pt = _NSA // 16
    wb_pt = _NS_ROWS // 16
    dump = _NS_ROWS

    def body(row_h, col_h, zeros_h, acc_h,
             row_v, col_v, pend_c, pend_il, pay, acc_sp, ps, sem2):
        cid = lax.axis_index("c")
        sid = lax.axis_index("s")
        wid = sid * 2 + cid
        i16 = lax.iota(jnp.int32, 16)
        eb = wid * _ET
        tru = jnp.ones((16,), jnp.bool_)

        def init_pend(z, _):
            pend_c[pl.ds(z * 16, 16)] = jnp.zeros((16,), jnp.int32) + dump
            pend_il[pl.ds(z * 16, 16)] = jnp.zeros((16,), jnp.float32)
            return 0

        lax.fori_loop(0, _PEND // 16, init_pend, 0)
        zrow = jnp.zeros((16,), jnp.float32)
        pltpu.sync_copy(zeros_h.at[pl.ds(0, _FLUSH), :], pay)

        def set_ones(g, _):
            plsc.store_scatter(pay, [g * 16 + i16, i16 * 0], zrow + 1.0,
                               mask=tru)
            return 0

        lax.fori_loop(0, _FLUSH // 16, set_ones, 0)

        pltpu.sync_copy(zeros_h, acc_sp.at[pl.ds(sid * rows_pt, rows_pt), :])
        plsc.subcore_barrier()
        ps[0] = 0

        def expand_il(nslots):
            def ex(g, _):
                il = pend_il[pl.ds(g * 16, 16)]
                plsc.store_scatter(pay, [g * 16 + i16, i16 * 0 + 1], il,
                                   mask=tru)
                return 0

            lax.fori_loop(0, nslots // 16, ex, 0)

        def flush3():
            expand_il(_FLUSH)
            ds_s = []
            for b in range(_NFB):
                ds_s.append(pltpu.async_copy(
                    pay.at[pl.ds(b * 128, 128), :],
                    acc_sp.at[pend_c.at[pl.ds(b * 128, 128)]], sem2,
                    add=True))
            for d in ds_s:
                d.wait()
            pend_c[pl.ds(0, 16)] = pend_c[pl.ds(_FLUSH, 16)]
            pend_il[pl.ds(0, 16)] = pend_il[pl.ds(_FLUSH, 16)]
            ps[0] = ps[0] - _FLUSH

        def blk(bk, _):
            off = eb + bk * _B1
            pltpu.sync_copy(row_h.at[pl.ds(off, _B1)], row_v)
            pltpu.sync_copy(col_h.at[pl.ds(off, _B1)], col_v)

            def grp(g, _):
                r16 = row_v[pl.ds(g * 16, 16)]
                c16 = col_v[pl.ds(g * 16, 16)]
                il = jnp.where(r16 == c16, 1.0, 0.0)
                pcnt = ps[0]
                plsc.store_compressed(pend_c.at[pl.ds(pcnt, 16)], c16,
                                      mask=tru)
                plsc.store_compressed(pend_il.at[pl.ds(pcnt, 16)], il,
                                      mask=tru)
                ps[0] = pcnt + 16

                @pl.when(ps[0] >= _FLUSH)
                def _():
                    flush3()

                return 0

            lax.fori_loop(0, _NGB, grp, 0)
            return 0

        lax.fori_loop(0, _NBK, blk, 0)

        pcnt = ps[0]

        def padg(g, _):
            idx = g * 16 + i16
            plsc.store_scatter(pend_c, [idx],
                               jnp.zeros((16,), jnp.int32) + dump,
                               mask=idx >= pcnt)
            plsc.store_scatter(pend_il, [idx], jnp.zeros((16,), jnp.float32),
                               mask=idx >= pcnt)
            return 0

        lax.fori_loop(0, _FLUSH // 16, padg, 0)
        expand_il(_FLUSH)
        nb = (pcnt + 127) // 128

        def dr(b, _):
            pltpu.sync_copy(pay.at[pl.ds(b * 128, 128), :],
                            acc_sp.at[pend_c.at[pl.ds(b * 128, 128)]],
                            add=True)
            return 0

        lax.fori_loop(0, nb, dr, 0)
        plsc.subcore_barrier()
        pltpu.sync_copy(
            acc_sp.at[pl.ds(sid * wb_pt, wb_pt), :],
            acc_h.at[cid, pl.ds(sid * wb_pt, wb_pt), :])

    mesh = plsc.VectorSubcoreMesh(core_axis_name="c", subcore_axis_name="s")
    return pl.kernel(
        body,
        out_type=[jax.ShapeDtypeStruct((2, _NS_ROWS, 8), jnp.float32)],
        mesh=mesh,
        compiler_params=pltpu.CompilerParams(needs_layout_passes=False,
                                             use_tc_tiling_on_sc=False),
        scratch_types=[
            pltpu.VMEM((_B1,), jnp.int32),
            pltpu.VMEM((_B1,), jnp.int32),
            pltpu.VMEM((_PEND,), jnp.int32),
            pltpu.VMEM((_PEND,), jnp.float32),
            pltpu.VMEM((_FLUSH, 8), jnp.float32),
            pltpu.VMEM_SHARED((_NSA, 8), jnp.float32),
            pltpu.SMEM((8,), jnp.int32),
            pltpu.SemaphoreType.DMA,
        ],
    )


_BR = 2000


def _ln_rows(x, eps=1e-5):
    mu = jnp.mean(x, axis=-1, keepdims=True)
    var = jnp.mean((x - mu) ** 2, axis=-1, keepdims=True)
    return (x - mu) / jnp.sqrt(var + eps)


def _ymat_kernel(h_ref, w_ref, dinv_ref, o_ref):
    o_ref[...] = dinv_ref[...] * jnp.dot(
        h_ref[...], w_ref[...], preferred_element_type=jnp.float32)


def _ymat(h, W, dinv):
    N, K = h.shape
    F = W.shape[1]
    return pl.pallas_call(
        _ymat_kernel,
        grid=(N // _BR,),
        in_specs=[
            pl.BlockSpec((_BR, K), lambda i: (i, 0)),
            pl.BlockSpec((K, F), lambda i: (0, 0)),
            pl.BlockSpec((_BR, 1), lambda i: (i, 0)),
        ],
        out_specs=pl.BlockSpec((_BR, F), lambda i: (i, 0)),
        out_shape=jax.ShapeDtypeStruct((N, F), jnp.float32),
    )(h, W, dinv)


def _comb_kernel(acc_ref, y_ref, dinv_ref, lw_ref, b_ref, o_ref):
    a = acc_ref[0] + acc_ref[1]
    o = dinv_ref[...] * (a + lw_ref[...] * y_ref[...]) + b_ref[...]
    o_ref[...] = jax.nn.relu(_ln_rows(o))


def _comb(acc, y, dinv, lw, b):
    N, F = y.shape
    return pl.pallas_call(
        _comb_kernel,
        grid=(N // _BR,),
        in_specs=[
            pl.BlockSpec((2, _BR, F), lambda i: (0, i, 0)),
            pl.BlockSpec((_BR, F), lambda i: (i, 0)),
            pl.BlockSpec((_BR, 1), lambda i: (i, 0)),
            pl.BlockSpec((_BR, 1), lambda i: (i, 0)),
            pl.BlockSpec((1, F), lambda i: (0, 0)),
        ],
        out_specs=pl.BlockSpec((_BR, F), lambda i: (i, 0)),
        out_shape=jax.ShapeDtypeStruct((N, F), jnp.float32),
    )(acc, y, dinv, lw, b.reshape(1, -1))


def _dinv_kernel(acc_ref, dinv_ref, lw_ref):
    cnt = acc_ref[0, :, 0] + acc_ref[1, :, 0]
    il = acc_ref[0, :, 1] + acc_ref[1, :, 1]
    lw = jnp.where(il > 0, 0.0, 1.0)
    deg = cnt + lw
    dinv_ref[...] = jnp.where(deg > 0, jax.lax.rsqrt(jnp.maximum(deg, 1e-30)),
                              0.0)[:, None]
    lw_ref[...] = lw[:, None]


def _dinv(acc):
    N = acc.shape[1]
    return pl.pallas_call(
        _dinv_kernel,
        grid=(N // _BR,),
        in_specs=[pl.BlockSpec((2, _BR, 8), lambda i: (0, i, 0))],
        out_specs=[
            pl.BlockSpec((_BR, 1), lambda i: (i, 0)),
            pl.BlockSpec((_BR, 1), lambda i: (i, 0)),
        ],
        out_shape=[
            jax.ShapeDtypeStruct((N, 1), jnp.float32),
            jax.ShapeDtypeStruct((N, 1), jnp.float32),
        ],
    )(acc)


def _proj_kernel(x_ref, a_ref, c_ref, o_ref):
    o_ref[...] = jax.nn.relu(_ln_rows(
        jnp.dot(x_ref[...], a_ref[...], preferred_element_type=jnp.float32)
        + c_ref[...]))


def _proj(x, A, c):
    N, K = x.shape
    F = A.shape[1]
    return pl.pallas_call(
        _proj_kernel,
        grid=(N // _BR,),
        in_specs=[
            pl.BlockSpec((_BR, K), lambda i: (i, 0)),
            pl.BlockSpec((K, F), lambda i: (0, 0)),
            pl.BlockSpec((1, F), lambda i: (0, 0)),
        ],
        out_specs=pl.BlockSpec((_BR, F), lambda i: (i, 0)),
        out_shape=jax.ShapeDtypeStruct((N, F), jnp.float32),
    )(x, A, c.reshape(1, -1))


def _value_kernel(res_ref, ind_ref, a3p_ref, a3i_ref, c3_ref, o_ref):
    res = res_ref[...]
    ind = ind_ref[...]
    o = (jnp.dot(res, a3p_ref[...], preferred_element_type=jnp.float32)
         + ind * a3i_ref[...] + c3_ref[...])
    o = jax.nn.relu(_ln_rows(o))
    keep = (res[:, 24:25] > 0) | (ind > 0.5)
    o_ref[...] = jnp.where(keep, o, 0.0)


def _value_head(res, ind, A3, c3):
    P = res.shape[0]
    F = A3.shape[1]
    a3p = jnp.zeros((_C32, F), jnp.float32).at[:24].set(A3[:24])
    a3i = A3[24].reshape(1, F)
    return pl.pallas_call(
        _value_kernel,
        grid=(P // _BR,),
        in_specs=[
            pl.BlockSpec((_BR, _C32), lambda i: (i, 0)),
            pl.BlockSpec((_BR, 1), lambda i: (i, 0)),
            pl.BlockSpec((_C32, F), lambda i: (0, 0)),
            pl.BlockSpec((1, F), lambda i: (0, 0)),
            pl.BlockSpec((1, F), lambda i: (0, 0)),
        ],
        out_specs=pl.BlockSpec((_BR, F), lambda i: (i, 0)),
        out_shape=jax.ShapeDtypeStruct((P, F), jnp.float32),
    )(res, ind.reshape(P, 1), a3p, a3i, c3.reshape(1, -1))


def _fold_head_kernel(ht_ref, hb_ref, vt_ref, vb_ref, a4_ref, c4_ref, o_ref):
    h = jnp.concatenate([ht_ref[...] + hb_ref[...],
                         vt_ref[...] + vb_ref[...]], axis=1)
    o_ref[...] = jax.nn.sigmoid(
        jnp.dot(h, a4_ref[...], preferred_element_type=jnp.float32)
        + c4_ref[...])


def _fold_head(hg, value, A4, c4):
    N, F1 = hg.shape
    F2 = value.shape[1]
    R = N // 2
    G = R // _BR
    return pl.pallas_call(
        _fold_head_kernel,
        grid=(G,),
        in_specs=[
            pl.BlockSpec((_BR, F1), lambda i: (i, 0)),
            pl.BlockSpec((_BR, F1), lambda i: (i + G, 0)),
            pl.BlockSpec((_BR, F2), lambda i: (i, 0)),
            pl.BlockSpec((_BR, F2), lambda i: (i + G, 0)),
            pl.BlockSpec((F1 + F2, A4.shape[1]), lambda i: (0, 0)),
            pl.BlockSpec((1, A4.shape[1]), lambda i: (0, 0)),
        ],
        out_specs=pl.BlockSpec((_BR, A4.shape[1]), lambda i: (i, 0)),
        out_shape=jax.ShapeDtypeStruct((R, A4.shape[1]), jnp.float32),
    )(hg, hg, value, value, A4, c4.reshape(1, -1))


def kernel(x, W10, b10, W11, b11, Wr0, br0, Wr1, br1, A1, c1, A2, c2, A3, c3, A4, c4, edge2, edge2_r, ei, pred_links, num_node, pos_mask):
    n = N_NODE
    row2 = edge2[0].astype(jnp.int32)
    col2 = edge2[1].astype(jnp.int32)
    row2r = edge2_r[0].astype(jnp.int32)
    col2r = edge2_r[1].astype(jnp.int32)

    skern = _make_gcn_struct_kernel()
    zeros8 = jnp.zeros((_NSA // 16, 8), jnp.float32)
    (acc_sa,) = skern(row2, col2, zeros8)
    (acc_sb,) = skern(row2r, col2r, zeros8)
    dinv_a, lw_a = _dinv(acc_sa)
    dinv_b, lw_b = _dinv(acc_sb)

    mkern = _make_gcn_msg_kernel()
    zeros32 = jnp.zeros((_CNA // 16, _C32), jnp.float32)
    h = x
    convs = [(W10, b10), (W11, b11)]
    convs_r = [(Wr0, br0), (Wr1, br1)]
    for i in range(2):
        ya = _ymat(h, convs[i][0], dinv_a)
        yb = _ymat(h, convs_r[i][0], dinv_b)
        (acc_a,) = mkern(row2, col2, ya, zeros32)
        (acc_b,) = mkern(row2r, col2r, yb, zeros32)
        ha = _comb(acc_a, ya, dinv_a, lw_a, convs[i][1])
        hb = _comb(acc_b, yb, dinv_b, lw_b, convs_r[i][1])
        h = ha + hb

    vm = jnp.where(pos_mask[:, None], x, 0.0)
    x1 = _proj(vm, A1, c1)
    x2 = _proj(vm, A2, c2)
    E = ei.shape[1]
    row = ei[0].astype(jnp.int32)
    col = ei[1].astype(jnp.int32)
    u = jnp.concatenate([x1[:E], jnp.ones((E, 1), jnp.float32)], axis=1)
    v = jnp.concatenate([x2[:E], jnp.ones((E, 1), jnp.float32)], axis=1)
    pi = pred_links[0].astype(jnp.int32)
    pk = pred_links[1].astype(jnp.int32)
    res, ind_f = _pair_products_sc(u, v, row, col, pi, pk, n)
    value = _value_head(res, ind_f, A3, c3)
    return _fold_head(h, value, A4, c4)


# speedup vs baseline: 80.4478x; 1.0699x over previous
import functools

import jax
import jax.numpy as jnp
from jax import lax
from jax.experimental import pallas as pl
from jax.experimental.pallas import tpu as pltpu
from jax.experimental.pallas import tpu_sc as plsc

N_NODE = 10000


def _ln(x, eps=1e-5):
    mu = jnp.mean(x, axis=-1, keepdims=True)
    var = jnp.mean((x - mu) ** 2, axis=-1, keepdims=True)
    return (x - mu) / jnp.sqrt(var + eps)


def _gcn_structure(edge, N):
    row = edge[0]
    col = edge[1]
    loop_idx = jnp.where(row == col, row, N)
    has_loop = jnp.zeros((N,), bool).at[loop_idx].set(True, mode='drop')
    loop_w = jnp.where(has_loop, 0.0, 1.0).astype(jnp.float32)
    deg = jnp.zeros((N,), jnp.float32).at[col].add(1.0) + loop_w
    dinv = jnp.where(deg > 0, 1.0 / jnp.sqrt(deg), 0.0)
    all_nodes = jnp.arange(N, dtype=row.dtype)
    row_f = jnp.concatenate([row, all_nodes])
    col_f = jnp.concatenate([col, all_nodes])
    norm_f = jnp.concatenate([dinv[row] * dinv[col], dinv * dinv * loop_w]).astype(jnp.float32)
    return row_f, col_f, norm_f


def _gcn(x, struct, W, b, N):
    row, col, norm = struct
    xw = x @ W
    msg = xw[row] * norm[:, None]
    return jax.ops.segment_sum(msg, col, num_segments=N) + b


def _pair_products_sparse(u, v, row, col, pi, pk, n):
    E = row.shape[0]
    code = row.astype(jnp.int32) * n + col.astype(jnp.int32)
    order = jnp.argsort(code)
    scode = code[order]
    scol = col.astype(jnp.int32)[order]
    us = u[order]
    vs = v[order]
    indptr = jnp.searchsorted(scode, jnp.arange(n + 1, dtype=jnp.int32) * n).astype(jnp.int32)
    deg = indptr[1:] - indptr[:-1]
    lo = indptr[pi]
    hi = indptr[pi + 1]
    P = pi.shape[0]

    def body(t, acc):
        e1 = jnp.minimum(lo + t, E - 1)
        active = (lo + t) < hi
        j = scol[e1]
        target = j * n + pk
        pos = jnp.clip(jnp.searchsorted(scode, target), 0, E - 1)
        found = active & (scode[pos] == target)
        prod = us[e1] * vs[pos] * found[:, None].astype(jnp.float32)
        return acc + prod

    maxdeg = jnp.max(deg)
    acc0 = jnp.zeros((P, u.shape[1]), jnp.float32)
    acc = jax.lax.fori_loop(0, maxdeg, body, acc0)
    return acc


_NW = 32
_CH = 256
_MB = 1024
_C32 = 32


def _make_pp_kernel(E, P, n):
    assert E % 2 == 0 and P % _CH == 0
    nchunk = P // _CH
    iptr_pad = ((n + 1 + 7) // 8) * 8
    i16 = lambda: lax.iota(jnp.int32, 16)

    def colval(colpk_v, e):
        word = plsc.load_gather(colpk_v, [e >> 1])
        return (word >> ((e & 1) << 4)) & 0xFFFF

    def lower_bound(colpk_v, l0, ln0, key, nrounds):
        def body(_, c):
            lo_, ln_ = c
            half = ln_ >> 1
            mid = lo_ + half
            cm = colval(colpk_v, jnp.minimum(mid, E - 1))
            less = cm < key
            lo2 = jnp.where(less, mid + 1, lo_)
            ln2 = jnp.where(less, ln_ - half - 1, half)
            return (lo2, ln2)

        lo_f, _ = lax.fori_loop(0, nrounds, body, (l0, ln0))
        return lo_f

    def body(colpk_h, iptr_h, pi_h, pk_h, us_h, vs_h, nr_h, res_h, ind_h,
             colpk_v, iptr_v, pi_v, pk_v, res_v, ind_v,
             m_pl, m_e1, m_e2, ubuf, vbuf, nr_v, cnt_s, sem1, sem2):
        wid = lax.axis_index("s") * 2 + lax.axis_index("c")
        pltpu.sync_copy(colpk_h, colpk_v)
        pltpu.sync_copy(iptr_h, iptr_v)
        pltpu.sync_copy(nr_h, nr_v)
        nrounds = jnp.max(nr_v[pl.ds(0, 16)])
        cnt_s[0] = 0
        zi = jnp.zeros((16,), jnp.int32)

        def init_mb(z, _):
            m_pl[pl.ds(z * 16, 16)] = zi
            m_e1[pl.ds(z * 16, 16)] = zi
            m_e2[pl.ds(z * 16, 16)] = zi
            return 0

        lax.fori_loop(0, _MB // 16, init_mb, 0)

        def flush():
            cnt = cnt_s[0]
            nb = (cnt + 127) // 128

            def fbody(b, _):
                off = b * 128
                d1 = pltpu.async_copy(us_h.at[m_e1.at[pl.ds(off, 128)]], ubuf, sem1)
                d2 = pltpu.async_copy(vs_h.at[m_e2.at[pl.ds(off, 128)]], vbuf, sem2)
                d1.wait()
                d2.wait()
                for g in range(8):
                    valid = (off + g * 16 + i16()) < cnt
                    plv = m_pl[pl.ds(off + g * 16, 16)]
                    midx = g * 16 + i16()

                    def cbody(c, _):
                        cs = jnp.zeros((16,), jnp.int32) + c
                        uv = plsc.load_gather(ubuf, [midx, cs])
                        vv = plsc.load_gather(vbuf, [midx, cs])
                        plsc.addupdate_scatter(res_v, [plv * _C32 + c], uv * vv,
                                               mask=valid)
                        return 0

                    lax.fori_loop(0, _C32, cbody, 0)
                return 0

            lax.fori_loop(0, nb, fbody, 0)
            cnt_s[0] = 0

        def chunk_body(it, _):
            cbase = (wid + it * _NW) * _CH
            pltpu.sync_copy(pi_h.at[pl.ds(cbase, _CH)], pi_v)
            pltpu.sync_copy(pk_h.at[pl.ds(cbase, _CH)], pk_v)

            def zbody(z, _):
                res_v[pl.ds(z * 16, 16)] = jnp.zeros((16,), jnp.float32)
                return 0

            lax.fori_loop(0, _CH * _C32 // 16, zbody, 0)

            def gloop(grp, _):
                lb = grp * 16
                vi = pi_v[pl.ds(lb, 16)]
                vk = pk_v[pl.ds(lb, 16)]
                lo = plsc.load_gather(iptr_v, [vi])
                hi = plsc.load_gather(iptr_v, [vi + 1])
                deg = hi - lo
                pos = lower_bound(colpk_v, lo, deg, vk, nrounds)
                fnd = ((pos < hi)
                       & (colval(colpk_v, jnp.minimum(pos, E - 1)) == vk))
                ind_v[pl.ds(lb, 16)] = jnp.where(fnd, 1.0, 0.0)
                dmax = jnp.max(deg)

                def tbody(t, _):
                    @pl.when(cnt_s[0] > _MB - 16)
                    def _():
                        flush()

                    e1 = jnp.minimum(lo + t, E - 1)
                    act = t < deg
                    j = colval(colpk_v, e1)
                    jlo = plsc.load_gather(iptr_v, [j])
                    jhi = plsc.load_gather(iptr_v, [j + 1])
                    ln0 = jnp.where(act, jhi - jlo, 0)
                    pos2 = lower_bound(colpk_v, jlo, ln0, vk, nrounds)
                    e2 = jnp.minimum(pos2, E - 1)
                    fd = act & (pos2 < jhi) & (colval(colpk_v, e2) == vk)
                    cnt = cnt_s[0]
                    plsc.store_compressed(m_pl.at[pl.ds(cnt, 16)], lb + i16(),
                                          mask=fd)
                    plsc.store_compressed(m_e1.at[pl.ds(cnt, 16)], e1, mask=fd)
                    plsc.store_compressed(m_e2.at[pl.ds(cnt, 16)], e2, mask=fd)
                    cnt_s[0] = cnt + jnp.sum(jnp.where(fd, 1, 0))
                    return 0

                lax.fori_loop(0, dmax, tbody, 0)
                return 0

            lax.fori_loop(0, _CH // 16, gloop, 0)
            flush()
            pltpu.sync_copy(res_v, res_h.at[pl.ds(cbase * _C32, _CH * _C32)])
            pltpu.sync_copy(ind_v, ind_h.at[pl.ds(cbase, _CH)])
            return 0

        niter = (nchunk - wid + _NW - 1) // _NW
        lax.fori_loop(0, niter, chunk_body, 0)

    mesh = plsc.VectorSubcoreMesh(core_axis_name="c", subcore_axis_name="s")
    return pl.kernel(
        body,
        out_type=[
            jax.ShapeDtypeStruct((P * _C32,), jnp.float32),
            jax.ShapeDtypeStruct((P,), jnp.float32),
        ],
        mesh=mesh,
        compiler_params=pltpu.CompilerParams(needs_layout_passes=False,
                                             use_tc_tiling_on_sc=False),
        scratch_types=[
            pltpu.VMEM((E // 2,), jnp.int32),
            pltpu.VMEM((iptr_pad,), jnp.int32),
            pltpu.VMEM((_CH,), jnp.int32),
            pltpu.VMEM((_CH,), jnp.int32),
            pltpu.VMEM((_CH * _C32,), jnp.float32),
            pltpu.VMEM((_CH,), jnp.float32),
            pltpu.VMEM((_MB,), jnp.int32),
            pltpu.VMEM((_MB,), jnp.int32),
            pltpu.VMEM((_MB,), jnp.int32),
            pltpu.VMEM((128, _C32), jnp.float32),
            pltpu.VMEM((128, _C32), jnp.float32),
            pltpu.VMEM((16,), jnp.int32),
            pltpu.SMEM((8,), jnp.int32),
            pltpu.SemaphoreType.DMA,
            pltpu.SemaphoreType.DMA,
        ],
    )


def _pair_products_sc(u, v, row, col, pi, pk, n):
    E = row.shape[0]
    P = pi.shape[0]
    code = row * n + col
    scode, order = lax.sort((code, lax.iota(jnp.int32, E)), num_keys=1,
                            is_stable=False)
    scol = scode % n
    us = jnp.zeros((E, _C32), jnp.float32).at[:, : u.shape[1]].set(u[order])
    vs = jnp.zeros((E, _C32), jnp.float32).at[:, : v.shape[1]].set(v[order])
    colpk = scol[0::2] | (scol[1::2] << 16)
    iptr_pad = ((n + 1 + 7) // 8) * 8
    iptr = jnp.full((iptr_pad,), E, jnp.int32).at[: n + 1].set(
        jnp.searchsorted(scode, jnp.arange(n + 1, dtype=jnp.int32) * n)
        .astype(jnp.int32))
    maxd = jnp.max(iptr[1 : n + 1] - iptr[:n])
    nrounds = jnp.where(
        maxd > 0,
        jnp.ceil(jnp.log2(maxd.astype(jnp.float32) + 1.0)).astype(jnp.int32),
        0) + 1
    nr_arr = jnp.zeros((16,), jnp.int32) + nrounds
    res_flat, ind = _make_pp_kernel(E, P, n)(colpk, iptr, pi, pk, us, vs,
                                             nr_arr)
    return res_flat.reshape(P, _C32), ind


_M2 = 2560000
_ET = _M2 // _NW
_B1 = 2000
_NGB = _B1 // 16
_NBK = _ET // _B1
_CN = 40704
_CNA = 40960
_NCHK = 4
_PEND = 1280
_FLUSH = 1152
_NFB = _FLUSH // 128
_NS_ROWS = 160000
_NSA = 160256


def _make_gcn_msg_kernel():
    rows_pt = _CNA // 16
    wb_pt = _CN // 16
    dump = _CN

    def body(row_h, col_h, y_h, zeros_h, acc_h,
             row_v, col_v, pend_r, pend_c, ybuf, acc_sp, ps, sem1, sem2):
        cid = lax.axis_index("c")
        sid = lax.axis_index("s")
        wid = sid * 2 + cid
        i16 = lax.iota(jnp.int32, 16)
        eb = wid * _ET

        def init_pend(z, _):
            pend_r[pl.ds(z * 16, 16)] = jnp.zeros((16,), jnp.int32)
            pend_c[pl.ds(z * 16, 16)] = jnp.zeros((16,), jnp.int32) + dump
            return 0

        lax.fori_loop(0, _PEND // 16, init_pend, 0)

        for ch in range(_NCHK):
            base = ch * _CN
            pltpu.sync_copy(zeros_h, acc_sp.at[pl.ds(sid * rows_pt, rows_pt), :])
            plsc.subcore_barrier()
            ps[0] = 0

            def flush3():
                ds_g = []
                for b in range(_NFB):
                    ds_g.append(pltpu.async_copy(
                        y_h.at[pend_r.at[pl.ds(b * 128, 128)]],
                        ybuf.at[pl.ds(b * 128, 128), :], sem1))
                for d in ds_g:
                    d.wait()
                ds_s = []
                for b in range(_NFB):
                    ds_s.append(pltpu.async_copy(
                        ybuf.at[pl.ds(b * 128, 128), :],
                        acc_sp.at[pend_c.at[pl.ds(b * 128, 128)]], sem2,
                        add=True))
                for d in ds_s:
                    d.wait()
                pend_r[pl.ds(0, 16)] = pend_r[pl.ds(_FLUSH, 16)]
                pend_c[pl.ds(0, 16)] = pend_c[pl.ds(_FLUSH, 16)]
                ps[0] = ps[0] - _FLUSH

            def blk(bk, _):
                off = eb + bk * _B1
                pltpu.sync_copy(row_h.at[pl.ds(off, _B1)], row_v)
                pltpu.sync_copy(col_h.at[pl.ds(off, _B1)], col_v)

                def grp(g, _):
                    r16 = row_v[pl.ds(g * 16, 16)]
                    c16 = col_v[pl.ds(g * 16, 16)]
                    m = (c16 >= base) & (c16 < base + _CN)
                    pcnt = ps[0]
                    plsc.store_compressed(pend_r.at[pl.ds(pcnt, 16)], r16,
                                          mask=m)
                    plsc.store_compressed(pend_c.at[pl.ds(pcnt, 16)],
                                          c16 - base, mask=m)
                    ps[0] = pcnt + jnp.sum(jnp.where(m, 1, 0))

                    @pl.when(ps[0] >= _FLUSH)
                    def _():
                        flush3()

                    return 0

                lax.fori_loop(0, _NGB, grp, 0)
                return 0

            lax.fori_loop(0, _NBK, blk, 0)

            pcnt = ps[0]

            def padg(g, _):
                idx = g * 16 + i16
                plsc.store_scatter(pend_c, [idx],
                                   jnp.zeros((16,), jnp.int32) + dump,
                                   mask=idx >= pcnt)
                return 0

            lax.fori_loop(0, _FLUSH // 16, padg, 0)
            nb = (pcnt + 127) // 128

            def dr(b, _):
                pltpu.async_copy(
                    y_h.at[pend_r.at[pl.ds(b * 128, 128)]],
                    ybuf.at[pl.ds(b * 128, 128), :], sem1).wait()
                pltpu.sync_copy(ybuf.at[pl.ds(b * 128, 128), :],
                                acc_sp.at[pend_c.at[pl.ds(b * 128, 128)]],
                                add=True)
                return 0

            lax.fori_loop(0, nb, dr, 0)
            ps[0] = 0
            plsc.subcore_barrier()
            pltpu.sync_copy(
                acc_sp.at[pl.ds(sid * wb_pt, wb_pt), :],
                acc_h.at[cid, pl.ds(base + sid * wb_pt, wb_pt), :])
            plsc.subcore_barrier()

    mesh = plsc.VectorSubcoreMesh(core_axis_name="c", subcore_axis_name="s")
    return pl.kernel(
        body,
        out_type=[jax.ShapeDtypeStruct((2, _NCHK * _CN, _C32), jnp.float32)],
        mesh=mesh,
        compiler_params=pltpu.CompilerParams(needs_layout_passes=False,
                                             use_tc_tiling_on_sc=False),
        scratch_types=[
            pltpu.VMEM((_B1,), jnp.int32),
            pltpu.VMEM((_B1,), jnp.int32),
            pltpu.VMEM((_PEND,), jnp.int32),
            pltpu.VMEM((_PEND,), jnp.int32),
            pltpu.VMEM((_FLUSH, _C32), jnp.float32),
            pltpu.VMEM_SHARED((_CNA, _C32), jnp.float32),
            pltpu.SMEM((8,), jnp.int32),
            pltpu.SemaphoreType.DMA,
            pltpu.SemaphoreType.DMA,
        ],
    )


def _make_gcn_struct_kernel():
    rows_pt = _NSA // 16
    wb_pt = _NS_ROWS // 16
    dump = _NS_ROWS

    def body(row_h, col_h, zeros_h, acc_h,
             row_v, col_v, pend_c, pend_il, pay, acc_sp, ps, sem2):
        cid = lax.axis_index("c")
        sid = lax.axis_index("s")
        wid = sid * 2 + cid
        i16 = lax.iota(jnp.int32, 16)
        eb = wid * _ET
        tru = jnp.ones((16,), jnp.bool_)

        def init_pend(z, _):
            pend_c[pl.ds(z * 16, 16)] = jnp.zeros((16,), jnp.int32) + dump
            pend_il[pl.ds(z * 16, 16)] = jnp.zeros((16,), jnp.float32)
            return 0

        lax.fori_loop(0, _PEND // 16, init_pend, 0)
        zrow = jnp.zeros((16,), jnp.float32)
        pltpu.sync_copy(zeros_h.at[pl.ds(0, _FLUSH), :], pay)

        def set_ones(g, _):
            plsc.store_scatter(pay, [g * 16 + i16, i16 * 0], zrow + 1.0,
                               mask=tru)
            return 0

        lax.fori_loop(0, _FLUSH // 16, set_ones, 0)

        pltpu.sync_copy(zeros_h, acc_sp.at[pl.ds(sid * rows_pt, rows_pt), :])
        plsc.subcore_barrier()
        ps[0] = 0

        def expand_il(nslots):
            def ex(g, _):
                il = pend_il[pl.ds(g * 16, 16)]
                plsc.store_scatter(pay, [g * 16 + i16, i16 * 0 + 1], il,
                                   mask=tru)
                return 0

            lax.fori_loop(0, nslots // 16, ex, 0)

        def flush3():
            expand_il(_FLUSH)
            ds_s = []
            for b in range(_NFB):
                ds_s.append(pltpu.async_copy(
                    pay.at[pl.ds(b * 128, 128), :],
                    acc_sp.at[pend_c.at[pl.ds(b * 128, 128)]], sem2,
                    add=True))
            for d in ds_s:
                d.wait()
            pend_c[pl.ds(0, 16)] = pend_c[pl.ds(_FLUSH, 16)]
            pend_il[pl.ds(0, 16)] = pend_il[pl.ds(_FLUSH, 16)]
            ps[0] = ps[0] - _FLUSH

        def blk(bk, _):
            off = eb + bk * _B1
            pltpu.sync_copy(row_h.at[pl.ds(off, _B1)], row_v)
            pltpu.sync_copy(col_h.at[pl.ds(off, _B1)], col_v)

            def grp(g, _):
                r16 = row_v[pl.ds(g * 16, 16)]
                c16 = col_v[pl.ds(g * 16, 16)]
                il = jnp.where(r16 == c16, 1.0, 0.0)
                pcnt = ps[0]
                plsc.store_compressed(pend_c.at[pl.ds(pcnt, 16)], c16,
                                      mask=tru)
                plsc.store_compressed(pend_il.at[pl.ds(pcnt, 16)], il,
                                      mask=tru)
                ps[0] = pcnt + 16

                @pl.when(ps[0] >= _FLUSH)
                def _():
                    flush3()

                return 0

            lax.fori_loop(0, _NGB, grp, 0)
            return 0

        lax.fori_loop(0, _NBK, blk, 0)

        pcnt = ps[0]

        def padg(g, _):
            idx = g * 16 + i16
            plsc.store_scatter(pend_c, [idx],
                               jnp.zeros((16,), jnp.int32) + dump,
                               mask=idx >= pcnt)
            plsc.store_scatter(pend_il, [idx], jnp.zeros((16,), jnp.float32),
                               mask=idx >= pcnt)
            return 0

        lax.fori_loop(0, _FLUSH // 16, padg, 0)
        expand_il(_FLUSH)
        nb = (pcnt + 127) // 128

        def dr(b, _):
            pltpu.sync_copy(pay.at[pl.ds(b * 128, 128), :],
                            acc_sp.at[pend_c.at[pl.ds(b * 128, 128)]],
                            add=True)
            return 0

        lax.fori_loop(0, nb, dr, 0)
        plsc.subcore_barrier()
        pltpu.sync_copy(
            acc_sp.at[pl.ds(sid * wb_pt, wb_pt), :],
            acc_h.at[cid, pl.ds(sid * wb_pt, wb_pt), :])

    mesh = plsc.VectorSubcoreMesh(core_axis_name="c", subcore_axis_name="s")
    return pl.kernel(
        body,
        out_type=[jax.ShapeDtypeStruct((2, _NS_ROWS, 8), jnp.float32)],
        mesh=mesh,
        compiler_params=pltpu.CompilerParams(needs_layout_passes=False,
                                             use_tc_tiling_on_sc=False),
        scratch_types=[
            pltpu.VMEM((_B1,), jnp.int32),
            pltpu.VMEM((_B1,), jnp.int32),
            pltpu.VMEM((_PEND,), jnp.int32),
            pltpu.VMEM((_PEND,), jnp.float32),
            pltpu.VMEM((_FLUSH, 8), jnp.float32),
            pltpu.VMEM_SHARED((_NSA, 8), jnp.float32),
            pltpu.SMEM((8,), jnp.int32),
            pltpu.SemaphoreType.DMA,
        ],
    )


_BR = 2000


def _ln_rows(x, eps=1e-5):
    mu = jnp.mean(x, axis=-1, keepdims=True)
    var = jnp.mean((x - mu) ** 2, axis=-1, keepdims=True)
    return (x - mu) / jnp.sqrt(var + eps)


def _ymat_kernel(h_ref, w_ref, dinv_ref, o_ref):
    o_ref[...] = dinv_ref[...] * jnp.dot(
        h_ref[...], w_ref[...], preferred_element_type=jnp.float32)


def _ymat(h, W, dinv):
    N, K = h.shape
    F = W.shape[1]
    return pl.pallas_call(
        _ymat_kernel,
        grid=(N // _BR,),
        in_specs=[
            pl.BlockSpec((_BR, K), lambda i: (i, 0)),
            pl.BlockSpec((K, F), lambda i: (0, 0)),
            pl.BlockSpec((_BR, 1), lambda i: (i, 0)),
        ],
        out_specs=pl.BlockSpec((_BR, F), lambda i: (i, 0)),
        out_shape=jax.ShapeDtypeStruct((N, F), jnp.float32),
    )(h, W, dinv)


def _comb_kernel(acc_ref, y_ref, dinv_ref, lw_ref, b_ref, o_ref):
    a = acc_ref[0] + acc_ref[1]
    o = dinv_ref[...] * (a + lw_ref[...] * y_ref[...]) + b_ref[...]
    o_ref[...] = jax.nn.relu(_ln_rows(o))


def _comb(acc, y, dinv, lw, b):
    N, F = y.shape
    return pl.pallas_call(
        _comb_kernel,
        grid=(N // _BR,),
        in_specs=[
            pl.BlockSpec((2, _BR, F), lambda i: (0, i, 0)),
            pl.BlockSpec((_BR, F), lambda i: (i, 0)),
            pl.BlockSpec((_BR, 1), lambda i: (i, 0)),
            pl.BlockSpec((_BR, 1), lambda i: (i, 0)),
            pl.BlockSpec((1, F), lambda i: (0, 0)),
        ],
        out_specs=pl.BlockSpec((_BR, F), lambda i: (i, 0)),
        out_shape=jax.ShapeDtypeStruct((N, F), jnp.float32),
    )(acc, y, dinv, lw, b.reshape(1, -1))


def _dinv_kernel(acc_ref, dinv_ref, lw_ref):
    cnt = acc_ref[0, :, 0] + acc_ref[1, :, 0]
    il = acc_ref[0, :, 1] + acc_ref[1, :, 1]
    lw = jnp.where(il > 0, 0.0, 1.0)
    deg = cnt + lw
    dinv_ref[...] = jnp.where(deg > 0, jax.lax.rsqrt(jnp.maximum(deg, 1e-30)),
                              0.0)[:, None]
    lw_ref[...] = lw[:, None]


def _dinv(acc):
    N = acc.shape[1]
    return pl.pallas_call(
        _dinv_kernel,
        grid=(N // _BR,),
        in_specs=[pl.BlockSpec((2, _BR, 8), lambda i: (0, i, 0))],
        out_specs=[
            pl.BlockSpec((_BR, 1), lambda i: (i, 0)),
            pl.BlockSpec((_BR, 1), lambda i: (i, 0)),
        ],
        out_shape=[
            jax.ShapeDtypeStruct((N, 1), jnp.float32),
            jax.ShapeDtypeStruct((N, 1), jnp.float32),
        ],
    )(acc)


def _proj_kernel(x_ref, a_ref, c_ref, o_ref):
    o_ref[...] = jax.nn.relu(_ln_rows(
        jnp.dot(x_ref[...], a_ref[...], preferred_element_type=jnp.float32)
        + c_ref[...]))


def _proj(x, A, c):
    N, K = x.shape
    F = A.shape[1]
    return pl.pallas_call(
        _proj_kernel,
        grid=(N // _BR,),
        in_specs=[
            pl.BlockSpec((_BR, K), lambda i: (i, 0)),
            pl.BlockSpec((K, F), lambda i: (0, 0)),
            pl.BlockSpec((1, F), lambda i: (0, 0)),
        ],
        out_specs=pl.BlockSpec((_BR, F), lambda i: (i, 0)),
        out_shape=jax.ShapeDtypeStruct((N, F), jnp.float32),
    )(x, A, c.reshape(1, -1))


def _value_kernel(res_ref, ind_ref, a3p_ref, a3i_ref, c3_ref, o_ref):
    res = res_ref[...]
    ind = ind_ref[...]
    o = (jnp.dot(res, a3p_ref[...], preferred_element_type=jnp.float32)
         + ind * a3i_ref[...] + c3_ref[...])
    o = jax.nn.relu(_ln_rows(o))
    keep = (res[:, 24:25] > 0) | (ind > 0.5)
    o_ref[...] = jnp.where(keep, o, 0.0)


def _value_head(res, ind, A3, c3):
    P = res.shape[0]
    F = A3.shape[1]
    a3p = jnp.zeros((_C32, F), jnp.float32).at[:24].set(A3[:24])
    a3i = A3[24].reshape(1, F)
    return pl.pallas_call(
        _value_kernel,
        grid=(P // _BR,),
        in_specs=[
            pl.BlockSpec((_BR, _C32), lambda i: (i, 0)),
            pl.BlockSpec((_BR, 1), lambda i: (i, 0)),
            pl.BlockSpec((_C32, F), lambda i: (0, 0)),
            pl.BlockSpec((1, F), lambda i: (0, 0)),
            pl.BlockSpec((1, F), lambda i: (0, 0)),
        ],
        out_specs=pl.BlockSpec((_BR, F), lambda i: (i, 0)),
        out_shape=jax.ShapeDtypeStruct((P, F), jnp.float32),
    )(res, ind.reshape(P, 1), a3p, a3i, c3.reshape(1, -1))


def _fold_head_kernel(ht_ref, hb_ref, vt_ref, vb_ref, a4_ref, c4_ref, o_ref):
    h = jnp.concatenate([ht_ref[...] + hb_ref[...],
                         vt_ref[...] + vb_ref[...]], axis=1)
    o_ref[...] = jax.nn.sigmoid(
        jnp.dot(h, a4_ref[...], preferred_element_type=jnp.float32)
        + c4_ref[...])


def _fold_head(hg, value, A4, c4):
    N, F1 = hg.shape
    F2 = value.shape[1]
    R = N // 2
    G = R // _BR
    return pl.pallas_call(
        _fold_head_kernel,
        grid=(G,),
        in_specs=[
            pl.BlockSpec((_BR, F1), lambda i: (i, 0)),
            pl.BlockSpec((_BR, F1), lambda i: (i + G, 0)),
            pl.BlockSpec((_BR, F2), lambda i: (i, 0)),
            pl.BlockSpec((_BR, F2), lambda i: (i + G, 0)),
            pl.BlockSpec((F1 + F2, A4.shape[1]), lambda i: (0, 0)),
            pl.BlockSpec((1, A4.shape[1]), lambda i: (0, 0)),
        ],
        out_specs=pl.BlockSpec((_BR, A4.shape[1]), lambda i: (i, 0)),
        out_shape=jax.ShapeDtypeStruct((R, A4.shape[1]), jnp.float32),
    )(hg, hg, value, value, A4, c4.reshape(1, -1))


def kernel(x, W10, b10, W11, b11, Wr0, br0, Wr1, br1, A1, c1, A2, c2, A3, c3, A4, c4, edge2, edge2_r, ei, pred_links, num_node, pos_mask):
    n = N_NODE
    row2 = edge2[0].astype(jnp.int32)
    col2 = edge2[1].astype(jnp.int32)
    row2r = edge2_r[0].astype(jnp.int32)
    col2r = edge2_r[1].astype(jnp.int32)

    skern = _make_gcn_struct_kernel()
    zeros8 = jnp.zeros((_NSA // 16, 8), jnp.float32)
    (acc_sa,) = skern(row2, col2, zeros8)
    (acc_sb,) = skern(row2r, col2r, zeros8)
    dinv_a, lw_a = _dinv(acc_sa)
    dinv_b, lw_b = _dinv(acc_sb)

    mkern = _make_gcn_msg_kernel()
    zeros32 = jnp.zeros((_CNA // 16, _C32), jnp.float32)
    h = x
    convs = [(W10, b10), (W11, b11)]
    convs_r = [(Wr0, br0), (Wr1, br1)]
    for i in range(2):
        ya = _ymat(h, convs[i][0], dinv_a)
        yb = _ymat(h, convs_r[i][0], dinv_b)
        (acc_a,) = mkern(row2, col2, ya, zeros32)
        (acc_b,) = mkern(row2r, col2r, yb, zeros32)
        ha = _comb(acc_a, ya, dinv_a, lw_a, convs[i][1])
        hb = _comb(acc_b, yb, dinv_b, lw_b, convs_r[i][1])
        h = ha + hb

    vm = jnp.where(pos_mask[:, None], x, 0.0)
    x1 = _proj(vm, A1, c1)
    x2 = _proj(vm, A2, c2)
    E = ei.shape[1]
    row = ei[0].astype(jnp.int32)
    col = ei[1].astype(jnp.int32)
    u = jnp.concatenate([x1[:E], jnp.ones((E, 1), jnp.float32)], axis=1)
    v = jnp.concatenate([x2[:E], jnp.ones((E, 1), jnp.float32)], axis=1)
    pi = pred_links[0].astype(jnp.int32)
    pk = pred_links[1].astype(jnp.int32)
    res, ind_f = _pair_products_sc(u, v, row, col, pi, pk, n)
    value = _value_head(res, ind_f, A3, c3)
    return _fold_head(h, value, A4, c4)
